# Initial kernel scaffold; baseline (speedup 1.0000x reference)
import jax
import jax.numpy as jnp
from jax import lax
from jax.experimental import pallas as pl
from jax.experimental.pallas import tpu as pltpu
from jax.experimental.pallas import tpu_sc as plsc

N = 10000
D = 128
E = 320000
NPAD = 10240
RPT = NPAD // 16
NTILE = 16
CH = 157
EPT = CH * 128
EP = NTILE * EPT

_F32 = jnp.float32
_I32 = jnp.int32


def _tc1_body(x_ref, w_ref, h_ref, s_ref, d_ref):
    y = jnp.dot(x_ref[0], w_ref[0], preferred_element_type=_F32)
    h_ref[0] = y[:, :16]
    s_ref[0, 0] = y[:, 16]
    d_ref[0, 0] = y[:, 17]


_tc1 = pl.pallas_call(
    _tc1_body,
    grid=(2,),
    in_specs=[
        pl.BlockSpec((1, NPAD, D), lambda t: (t, 0, 0)),
        pl.BlockSpec((1, D, 24), lambda t: (t, 0, 0)),
    ],
    out_specs=[
        pl.BlockSpec((1, NPAD, 16), lambda t: (t, 0, 0)),
        pl.BlockSpec((1, 1, NPAD), lambda t: (t, 0, 0)),
        pl.BlockSpec((1, 1, NPAD), lambda t: (t, 0, 0)),
    ],
    out_shape=[
        jax.ShapeDtypeStruct((2, NPAD, 16), _F32),
        jax.ShapeDtypeStruct((2, 1, NPAD), _F32),
        jax.ShapeDtypeStruct((2, 1, NPAD), _F32),
    ],
)


def _tc2_body(agg_ref, w_ref, h_ref, s_ref, d_ref):
    x2 = jnp.maximum(agg_ref[0], 0.0)
    y = jnp.dot(x2, w_ref[0], preferred_element_type=_F32)
    h_ref[0] = y[:, :2]
    s_ref[0, 0] = y[:, 2]
    d_ref[0, 0] = y[:, 3]


_tc2 = pl.pallas_call(
    _tc2_body,
    grid=(2,),
    in_specs=[
        pl.BlockSpec((1, NPAD, 16), lambda t: (1 - t, 0, 0)),
        pl.BlockSpec((1, 16, 8), lambda t: (t, 0, 0)),
    ],
    out_specs=[
        pl.BlockSpec((1, NPAD, 2), lambda t: (t, 0, 0)),
        pl.BlockSpec((1, 1, NPAD), lambda t: (t, 0, 0)),
        pl.BlockSpec((1, 1, NPAD), lambda t: (t, 0, 0)),
    ],
    out_shape=[
        jax.ShapeDtypeStruct((2, NPAD, 2), _F32),
        jax.ShapeDtypeStruct((2, 1, NPAD), _F32),
        jax.ShapeDtypeStruct((2, 1, NPAD), _F32),
    ],
)


def _make_sc(h_dim):
    mesh = plsc.VectorSubcoreMesh(core_axis_name="c", subcore_axis_name="s")
    if h_dim == 16:
        out_type = jax.ShapeDtypeStruct((2, NPAD, 16), _F32)
        scratch = [
            pltpu.VMEM((CH, 128), _I32),
            pltpu.VMEM((CH, 128), _I32),
            pltpu.VMEM((CH, 128), _F32),
            pltpu.VMEM((CH, 128), _F32),
            pltpu.VMEM((NPAD,), _F32),
            pltpu.VMEM((NPAD,), _F32),
            pltpu.VMEM((RPT, 16), _F32),
            pltpu.VMEM((128, 16), _F32),
            pltpu.VMEM((16,), _F32),
            pltpu.VMEM((16, 16), _F32),
            pltpu.VMEM((16,), _F32),
            pltpu.VMEM((RPT,), _F32),
            pltpu.MemorySpace.VMEM_SHARED((NPAD, 16), _F32),
            pltpu.MemorySpace.VMEM_SHARED((NPAD, 16), _F32),
            pltpu.MemorySpace.VMEM_SHARED((NPAD,), _F32),
            pltpu.MemorySpace.VMEM_SHARED((16, 16), _F32),
        ]
    else:
        out_type = jax.ShapeDtypeStruct((2, NPAD * 2), _F32)
        scratch = [
            pltpu.VMEM((CH, 128), _I32),
            pltpu.VMEM((CH, 128), _I32),
            pltpu.VMEM((CH, 128), _F32),
            pltpu.VMEM((CH, 128), _F32),
            pltpu.VMEM((NPAD,), _F32),
            pltpu.VMEM((NPAD,), _F32),
            pltpu.VMEM((NPAD, 2), _F32),
            pltpu.VMEM((2, 128), _F32),
            pltpu.VMEM((2, 128), _I32),
            pltpu.VMEM((2 * RPT,), _F32),
            pltpu.VMEM((16,), _F32),
            pltpu.VMEM((16, 16), _F32),
            pltpu.VMEM((16,), _F32),
            pltpu.VMEM((RPT,), _F32),
            pltpu.MemorySpace.VMEM_SHARED((NPAD * 2,), _F32),
            pltpu.MemorySpace.VMEM_SHARED((NPAD,), _F32),
            pltpu.MemorySpace.VMEM_SHARED((16, 16), _F32),
        ]

    def body(h_hbm, s_hbm, d_hbm, src_hbm, dst_hbm, ew_hbm, b_hbm, out_hbm,
             *scr):
        if h_dim == 16:
            (srcv, dstv, eww, work, sbuf, dbuf, hstage, rows, maxv, maxall,
             bbuf, zv, h_sp, out_sp, den_sp, max_sp) = scr
        else:
            (srcv, dstv, eww, work, sbuf, dbuf, h2d, vals, dstx2, ostage,
             maxv, maxall, bbuf, zv, out_sp, den_sp, max_sp) = scr
        w = lax.axis_index("s")
        rel = lax.axis_index("c")
        r0 = w * RPT
        zero16 = jnp.zeros((16,), _F32)

        pltpu.sync_copy(src_hbm.at[rel, w], srcv)
        pltpu.sync_copy(dst_hbm.at[rel, w], dstv)
        pltpu.sync_copy(ew_hbm.at[rel, w], eww)
        pltpu.sync_copy(s_hbm.at[rel, 0], sbuf)
        pltpu.sync_copy(d_hbm.at[1 - rel, 0], dbuf)
        pltpu.sync_copy(b_hbm.at[rel], bbuf)

        def _zv_row(i, _):
            zv[pl.ds(i * 16, 16)] = zero16
            return 0
        lax.fori_loop(0, RPT // 16, _zv_row, 0)
        pltpu.sync_copy(zv.at[pl.ds(0, RPT)], den_sp.at[pl.ds(r0, RPT)])

        if h_dim == 16:
            pltpu.sync_copy(h_hbm.at[rel, pl.ds(r0, RPT)], hstage)
            pltpu.sync_copy(hstage, h_sp.at[pl.ds(r0, RPT)])

            def _z_row(r, _):
                hstage[r, :] = zero16
                return 0
            lax.fori_loop(0, RPT, _z_row, 0)
            pltpu.sync_copy(hstage, out_sp.at[pl.ds(r0, RPT)])
        else:
            pltpu.sync_copy(h_hbm.at[rel], h2d)

            def _z_row2(i, _):
                ostage[pl.ds(i * 16, 16)] = zero16
                return 0
            lax.fori_loop(0, (2 * RPT) // 16, _z_row2, 0)
            pltpu.sync_copy(ostage, out_sp.at[pl.ds(2 * r0, 2 * RPT)])

        neg = jnp.full((16,), -3.0e38, _F32)

        def _p1_row(cc, runmax):
            def _p1_k(k, rm):
                si = srcv[cc, pl.ds(k * 16, 16)]
                di = dstv[cc, pl.ds(k * 16, 16)]
                sv = plsc.load_gather(sbuf, [si])
                dv = plsc.load_gather(dbuf, [di])
                logit = sv + dv
                logit = jnp.where(logit > 0.0, logit, 0.2 * logit)
                work[cc, pl.ds(k * 16, 16)] = logit
                return jnp.maximum(rm, logit)
            return lax.fori_loop(0, 8, _p1_k, runmax)

        runmax = lax.fori_loop(0, CH, _p1_row, neg)
        maxv[:] = jnp.broadcast_to(jnp.max(runmax), (16,))
        pltpu.sync_copy(maxv, max_sp.at[w])
        plsc.subcore_barrier()

        pltpu.sync_copy(max_sp, maxall)

        def _mred(i, mm):
            return jnp.maximum(mm, maxall[i, :])
        m_glob = jnp.max(lax.fori_loop(0, 16, _mred, neg))

        def _p2_row(cc, _):
            def _p2_k(k, __):
                logit = work[cc, pl.ds(k * 16, 16)]
                work[cc, pl.ds(k * 16, 16)] = jnp.exp(logit - m_glob)
                return 0
            lax.fori_loop(0, 8, _p2_k, 0)
            pltpu.sync_copy(work.at[cc], den_sp.at[dstv.at[cc]], add=True)
            return 0
        lax.fori_loop(0, CH, _p2_row, 0)
        plsc.subcore_barrier()
        pltpu.sync_copy(den_sp, sbuf)

        iota = lax.iota(_I32, 16)
        zi = iota * 0
        oi = zi + 1

        def _p34_row(cc, _):
            def _p3_k(k, __):
                ex = work[cc, pl.ds(k * 16, 16)]
                di = dstv[cc, pl.ds(k * 16, 16)]
                den = plsc.load_gather(sbuf, [di])
                alpha = ex / (den + 1e-16)
                work[cc, pl.ds(k * 16, 16)] = (
                    alpha * eww[cc, pl.ds(k * 16, 16)])
                return 0
            lax.fori_loop(0, 8, _p3_k, 0)
            if h_dim == 16:
                pltpu.sync_copy(h_sp.at[srcv.at[cc]], rows)

                def _scale_e(e, __):
                    c = work[cc, e]
                    rows[e, :] = rows[e, :] * c
                    return 0
                lax.fori_loop(0, 128, _scale_e, 0)
                pltpu.sync_copy(rows, out_sp.at[dstv.at[cc]], add=True)
            else:
                def _h2_k(k, __):
                    si = srcv[cc, pl.ds(k * 16, 16)]
                    di = dstv[cc, pl.ds(k * 16, 16)]
                    co = work[cc, pl.ds(k * 16, 16)]
                    vals[0, pl.ds(k * 16, 16)] = \
                        plsc.load_gather(h2d, [si, zi]) * co
                    vals[1, pl.ds(k * 16, 16)] = \
                        plsc.load_gather(h2d, [si, oi]) * co
                    dstx2[0, pl.ds(k * 16, 16)] = di * 2
                    dstx2[1, pl.ds(k * 16, 16)] = di * 2 + 1
                    return 0
                lax.fori_loop(0, 8, _h2_k, 0)
                pltpu.sync_copy(vals.at[0], out_sp.at[dstx2.at[0]], add=True)
                pltpu.sync_copy(vals.at[1], out_sp.at[dstx2.at[1]], add=True)
            return 0
        lax.fori_loop(0, CH, _p34_row, 0)
        plsc.subcore_barrier()

        bv = bbuf[:]
        if h_dim == 16:
            pltpu.sync_copy(out_sp.at[pl.ds(r0, RPT)], hstage)

            def _b_row(r, _):
                hstage[r, :] = hstage[r, :] + bv
                return 0
            lax.fori_loop(0, RPT, _b_row, 0)
            pltpu.sync_copy(hstage, out_hbm.at[rel, pl.ds(r0, RPT)])
        else:
            o0 = 2 * r0
            pltpu.sync_copy(out_sp.at[pl.ds(o0, 2 * RPT)], ostage)

            def _b_row2(i, _):
                ostage[pl.ds(i * 16, 16)] = ostage[pl.ds(i * 16, 16)] + bv
                return 0
            lax.fori_loop(0, (2 * RPT) // 16, _b_row2, 0)
            pltpu.sync_copy(ostage, out_hbm.at[rel, pl.ds(o0, 2 * RPT)])

    return pl.kernel(body, out_type=out_type, mesh=mesh,
                     scratch_types=scratch)


_sc_l1 = _make_sc(16)
_sc_l2 = _make_sc(2)


def _pad_edges(ei, ew):
    src = jnp.concatenate([ei[0], jnp.full((EP - E,), N, _I32)])
    dst = jnp.concatenate([ei[1], jnp.full((EP - E,), N, _I32)])
    ewp = jnp.concatenate([ew, jnp.zeros((EP - E,), _F32)])
    return (src.reshape(NTILE, CH, 128), dst.reshape(NTILE, CH, 128),
            ewp.reshape(NTILE, CH, 128))


def kernel(x_user, x_badge, edge_index_u2b, edge_index_b2u,
           edge_weight_u2b, edge_weight_b2u,
           W1ub_s, W1ub_d, a1ub_s, a1ub_d, b1ub,
           W1bu_s, W1bu_d, a1bu_s, a1bu_d, b1bu,
           W2ub_s, W2ub_d, a2ub_s, a2ub_d, b2ub,
           W2bu_s, W2bu_d, a2bu_s, a2bu_d, b2bu):
    pad = ((0, NPAD - N), (0, 0))
    x_st = jnp.stack([jnp.pad(x_user, pad), jnp.pad(x_badge, pad)])

    def _wcat1(ws, a_s, wd_other, a_d_other):
        return jnp.concatenate(
            [ws, (ws @ a_s)[:, None], (wd_other @ a_d_other)[:, None],
             jnp.zeros((D, 6), _F32)], axis=1)

    w1cat = jnp.stack([_wcat1(W1ub_s, a1ub_s, W1bu_d, a1bu_d),
                       _wcat1(W1bu_s, a1bu_s, W1ub_d, a1ub_d)])

    su, du, eu = _pad_edges(edge_index_u2b, edge_weight_u2b)
    sb, db, eb = _pad_edges(edge_index_b2u, edge_weight_b2u)
    src_st = jnp.stack([su, sb])
    dst_st = jnp.stack([du, db])
    ew_st = jnp.stack([eu, eb])

    bias1 = jnp.stack([b1ub, b1bu])
    bias2 = jnp.stack([jnp.tile(b2ub, 8), jnp.tile(b2bu, 8)])

    h1, s1, d1 = _tc1(x_st, w1cat)
    agg1 = _sc_l1(h1, s1, d1, src_st, dst_st, ew_st, bias1)

    def _wcat2(ws, a_s, wd_other, a_d_other):
        return jnp.concatenate(
            [ws, (ws @ a_s)[:, None], (wd_other @ a_d_other)[:, None],
             jnp.zeros((16, 4), _F32)], axis=1)

    w2cat = jnp.stack([_wcat2(W2ub_s, a2ub_s, W2bu_d, a2bu_d),
                       _wcat2(W2bu_s, a2bu_s, W2ub_d, a2ub_d)])

    h2, s2, d2 = _tc2(agg1, w2cat)
    agg2 = _sc_l2(h2, s2, d2, src_st, dst_st, ew_st, bias2)

    badge2 = agg2[0].reshape(NPAD, 2)[:N]
    user2 = agg2[1].reshape(NPAD, 2)[:N]
    return (user2, badge2)


# speedup vs baseline: 66.1538x; 66.1538x over previous
import functools

import jax
import jax.numpy as jnp
from jax import lax
from jax.experimental import pallas as pl
from jax.experimental.pallas import tpu as pltpu
from jax.experimental.pallas import tpu_sc as plsc

N = 10000
D = 128
E = 320000
NPAD = 10240
RPT = NPAD // 16
NTILE = 16
CH = 157
EPT = CH * 128
EP = NTILE * EPT

_F32 = jnp.float32
_I32 = jnp.int32


def _tc1_body(x_ref, w_ref, h_ref, s_ref, d_ref):
    y = jnp.dot(x_ref[0], w_ref[0], preferred_element_type=_F32)
    h_ref[0] = y[:, :16]
    s_ref[0, 0] = y[:, 16]
    d_ref[0, 0] = y[:, 17]


_tc1 = pl.pallas_call(
    _tc1_body,
    grid=(2,),
    in_specs=[
        pl.BlockSpec((1, NPAD, D), lambda t: (t, 0, 0)),
        pl.BlockSpec((1, D, 24), lambda t: (t, 0, 0)),
    ],
    out_specs=[
        pl.BlockSpec((1, NPAD, 16), lambda t: (t, 0, 0)),
        pl.BlockSpec((1, 1, NPAD), lambda t: (t, 0, 0)),
        pl.BlockSpec((1, 1, NPAD), lambda t: (t, 0, 0)),
    ],
    out_shape=[
        jax.ShapeDtypeStruct((2, NPAD, 16), _F32),
        jax.ShapeDtypeStruct((2, 1, NPAD), _F32),
        jax.ShapeDtypeStruct((2, 1, NPAD), _F32),
    ],
)


def _tc2_body(agg_ref, w_ref, h_ref, s_ref, d_ref):
    x2 = jnp.maximum(agg_ref[0], 0.0)
    y = jnp.dot(x2, w_ref[0], preferred_element_type=_F32)
    h_ref[0] = y[:, :2]
    s_ref[0, 0] = y[:, 2]
    d_ref[0, 0] = y[:, 3]


_tc2 = pl.pallas_call(
    _tc2_body,
    grid=(2,),
    in_specs=[
        pl.BlockSpec((1, NPAD, 16), lambda t: (1 - t, 0, 0)),
        pl.BlockSpec((1, 16, 8), lambda t: (t, 0, 0)),
    ],
    out_specs=[
        pl.BlockSpec((1, NPAD, 2), lambda t: (t, 0, 0)),
        pl.BlockSpec((1, 1, NPAD), lambda t: (t, 0, 0)),
        pl.BlockSpec((1, 1, NPAD), lambda t: (t, 0, 0)),
    ],
    out_shape=[
        jax.ShapeDtypeStruct((2, NPAD, 2), _F32),
        jax.ShapeDtypeStruct((2, 1, NPAD), _F32),
        jax.ShapeDtypeStruct((2, 1, NPAD), _F32),
    ],
)


@functools.lru_cache(maxsize=None)
def _make_sc(h_dim):
    mesh = plsc.VectorSubcoreMesh(core_axis_name="c", subcore_axis_name="s",
                                  num_cores=2, num_subcores=NTILE)
    if h_dim == 16:
        out_type = jax.ShapeDtypeStruct((2, NPAD, 16), _F32)
        scratch = [
            pltpu.VMEM((CH, 128), _I32),
            pltpu.VMEM((CH, 128), _I32),
            pltpu.VMEM((128,), _F32),
            pltpu.VMEM((CH, 128), _F32),
            pltpu.VMEM((NPAD,), _F32),
            pltpu.VMEM((NPAD,), _F32),
            pltpu.VMEM((RPT, 16), _F32),
            pltpu.VMEM((128, 16), _F32),
            pltpu.VMEM((16,), _F32),
            pltpu.VMEM((16, 16), _F32),
            pltpu.VMEM((16,), _F32),
            pltpu.VMEM((RPT,), _F32),
            pltpu.MemorySpace.VMEM_SHARED((NPAD, 16), _F32),
            pltpu.MemorySpace.VMEM_SHARED((NPAD, 16), _F32),
            pltpu.MemorySpace.VMEM_SHARED((NPAD,), _F32),
            pltpu.MemorySpace.VMEM_SHARED((16, 16), _F32),
        ]
    else:
        out_type = jax.ShapeDtypeStruct((2, NPAD * 2), _F32)
        scratch = [
            pltpu.VMEM((CH, 128), _I32),
            pltpu.VMEM((CH, 128), _I32),
            pltpu.VMEM((128,), _F32),
            pltpu.VMEM((CH, 128), _F32),
            pltpu.VMEM((NPAD,), _F32),
            pltpu.VMEM((NPAD,), _F32),
            pltpu.VMEM((NPAD * 2,), _F32),
            pltpu.VMEM((2, 128), _F32),
            pltpu.VMEM((2, 128), _I32),
            pltpu.VMEM((2 * RPT,), _F32),
            pltpu.VMEM((16,), _F32),
            pltpu.VMEM((16, 16), _F32),
            pltpu.VMEM((16,), _F32),
            pltpu.VMEM((RPT,), _F32),
            pltpu.MemorySpace.VMEM_SHARED((NPAD * 2,), _F32),
            pltpu.MemorySpace.VMEM_SHARED((NPAD,), _F32),
            pltpu.MemorySpace.VMEM_SHARED((16, 16), _F32),
        ]

    def body(h_hbm, s_hbm, d_hbm, src_hbm, dst_hbm, ew_hbm, b_hbm, out_hbm,
             *scr):
        if h_dim == 16:
            (srcv, dstv, ewrow, work, sbuf, dbuf, hstage, rows, maxv, maxall,
             bbuf, zv, h_sp, out_sp, den_sp, max_sp) = scr
        else:
            (srcv, dstv, ewrow, work, sbuf, dbuf, h2d, vals, dstx2, ostage,
             maxv, maxall, bbuf, zv, out_sp, den_sp, max_sp) = scr
        w = lax.axis_index("s")
        rel = lax.axis_index("c")
        r0 = w * RPT
        zero16 = jnp.zeros((16,), _F32)

        pltpu.sync_copy(src_hbm.at[rel, w], srcv)
        pltpu.sync_copy(dst_hbm.at[rel, w], dstv)
        pltpu.sync_copy(s_hbm.at[rel, 0], sbuf)
        pltpu.sync_copy(d_hbm.at[1 - rel, 0], dbuf)
        pltpu.sync_copy(b_hbm.at[rel], bbuf)

        def _zv_row(i, _):
            zv[pl.ds(i * 16, 16)] = zero16
            return 0
        lax.fori_loop(0, RPT // 16, _zv_row, 0)
        pltpu.sync_copy(zv.at[pl.ds(0, RPT)], den_sp.at[pl.ds(r0, RPT)])

        if h_dim == 16:
            pltpu.sync_copy(h_hbm.at[rel, pl.ds(r0, RPT)], hstage)
            pltpu.sync_copy(hstage, h_sp.at[pl.ds(r0, RPT)])

            def _z_row(r, _):
                hstage[r, :] = zero16
                return 0
            lax.fori_loop(0, RPT, _z_row, 0)
            pltpu.sync_copy(hstage, out_sp.at[pl.ds(r0, RPT)])
        else:
            pltpu.sync_copy(h_hbm.at[rel], h2d)

            def _z_row2(i, _):
                ostage[pl.ds(i * 16, 16)] = zero16
                return 0
            lax.fori_loop(0, (2 * RPT) // 16, _z_row2, 0)
            pltpu.sync_copy(ostage, out_sp.at[pl.ds(2 * r0, 2 * RPT)])

        neg = jnp.full((16,), -3.0e38, _F32)

        def _p1_row(cc, runmax):
            def _p1_k(k, rm):
                si = srcv[cc, pl.ds(k * 16, 16)]
                di = dstv[cc, pl.ds(k * 16, 16)]
                sv = plsc.load_gather(sbuf, [si])
                dv = plsc.load_gather(dbuf, [di])
                logit = sv + dv
                logit = jnp.where(logit > 0.0, logit, 0.2 * logit)
                work[cc, pl.ds(k * 16, 16)] = logit
                return jnp.maximum(rm, logit)
            return lax.fori_loop(0, 8, _p1_k, runmax)

        runmax = lax.fori_loop(0, CH, _p1_row, neg)
        maxv[:] = jnp.broadcast_to(jnp.max(runmax), (16,))
        pltpu.sync_copy(maxv, max_sp.at[w])
        plsc.subcore_barrier()

        pltpu.sync_copy(max_sp, maxall)

        def _mred(i, mm):
            return jnp.maximum(mm, maxall[i, :])
        m_glob = jnp.max(lax.fori_loop(0, 16, _mred, neg))

        def _p2_row(cc, _):
            def _p2_k(k, __):
                logit = work[cc, pl.ds(k * 16, 16)]
                work[cc, pl.ds(k * 16, 16)] = jnp.exp(logit - m_glob)
                return 0
            lax.fori_loop(0, 8, _p2_k, 0)
            pltpu.sync_copy(work.at[cc], den_sp.at[dstv.at[cc]], add=True)
            return 0
        lax.fori_loop(0, CH, _p2_row, 0)
        plsc.subcore_barrier()
        pltpu.sync_copy(den_sp, sbuf)

        iota = lax.iota(_I32, 16)
        zi = iota * 0
        oi = zi + 1

        def _p34_row(cc, _):
            pltpu.sync_copy(ew_hbm.at[rel, w, cc], ewrow)

            def _p3_k(k, __):
                ex = work[cc, pl.ds(k * 16, 16)]
                di = dstv[cc, pl.ds(k * 16, 16)]
                den = plsc.load_gather(sbuf, [di])
                alpha = ex / (den + 1e-16)
                work[cc, pl.ds(k * 16, 16)] = (
                    alpha * ewrow[pl.ds(k * 16, 16)])
                return 0
            lax.fori_loop(0, 8, _p3_k, 0)
            if h_dim == 16:
                pltpu.sync_copy(h_sp.at[srcv.at[cc]], rows)

                def _scale_e(e, __):
                    ce = plsc.load_gather(work.at[cc], [zi + e])
                    rows[e, :] = rows[e, :] * ce
                    return 0
                lax.fori_loop(0, 128, _scale_e, 0)
                pltpu.sync_copy(rows, out_sp.at[dstv.at[cc]], add=True)
            else:
                def _h2_k(k, __):
                    si = srcv[cc, pl.ds(k * 16, 16)]
                    di = dstv[cc, pl.ds(k * 16, 16)]
                    co = work[cc, pl.ds(k * 16, 16)]
                    si2 = si * 2
                    vals[0, pl.ds(k * 16, 16)] = \
                        plsc.load_gather(h2d, [si2]) * co
                    vals[1, pl.ds(k * 16, 16)] = \
                        plsc.load_gather(h2d, [si2 + 1]) * co
                    dstx2[0, pl.ds(k * 16, 16)] = di * 2
                    dstx2[1, pl.ds(k * 16, 16)] = di * 2 + 1
                    return 0
                lax.fori_loop(0, 8, _h2_k, 0)
                pltpu.sync_copy(vals.at[0], out_sp.at[dstx2.at[0]], add=True)
                pltpu.sync_copy(vals.at[1], out_sp.at[dstx2.at[1]], add=True)
            return 0
        lax.fori_loop(0, CH, _p34_row, 0)
        plsc.subcore_barrier()

        bv = bbuf[:]
        if h_dim == 16:
            pltpu.sync_copy(out_sp.at[pl.ds(r0, RPT)], hstage)

            def _b_row(r, _):
                hstage[r, :] = hstage[r, :] + bv
                return 0
            lax.fori_loop(0, RPT, _b_row, 0)
            pltpu.sync_copy(hstage, out_hbm.at[rel, pl.ds(r0, RPT)])
        else:
            o0 = 2 * r0
            pltpu.sync_copy(out_sp.at[pl.ds(o0, 2 * RPT)], ostage)

            def _b_row2(i, _):
                ostage[pl.ds(i * 16, 16)] = ostage[pl.ds(i * 16, 16)] + bv
                return 0
            lax.fori_loop(0, (2 * RPT) // 16, _b_row2, 0)
            pltpu.sync_copy(ostage, out_hbm.at[rel, pl.ds(o0, 2 * RPT)])

    return pl.kernel(
        body, out_type=out_type, mesh=mesh, scratch_types=scratch,
        compiler_params=pltpu.CompilerParams(needs_layout_passes=False,
                                             use_tc_tiling_on_sc=False))


def _pad_edges(ei, ew):
    src = jnp.concatenate([ei[0], jnp.full((EP - E,), N, _I32)])
    dst = jnp.concatenate([ei[1], jnp.full((EP - E,), N, _I32)])
    ewp = jnp.concatenate([ew, jnp.zeros((EP - E,), _F32)])
    return (src.reshape(NTILE, CH, 128), dst.reshape(NTILE, CH, 128),
            ewp.reshape(NTILE, CH, 128))


def kernel(x_user, x_badge, edge_index_u2b, edge_index_b2u,
           edge_weight_u2b, edge_weight_b2u,
           W1ub_s, W1ub_d, a1ub_s, a1ub_d, b1ub,
           W1bu_s, W1bu_d, a1bu_s, a1bu_d, b1bu,
           W2ub_s, W2ub_d, a2ub_s, a2ub_d, b2ub,
           W2bu_s, W2bu_d, a2bu_s, a2bu_d, b2bu):
    pad = ((0, NPAD - N), (0, 0))
    x_st = jnp.stack([jnp.pad(x_user, pad), jnp.pad(x_badge, pad)])

    def _wcat1(ws, a_s, wd_other, a_d_other):
        return jnp.concatenate(
            [ws, (ws @ a_s)[:, None], (wd_other @ a_d_other)[:, None],
             jnp.zeros((D, 6), _F32)], axis=1)

    w1cat = jnp.stack([_wcat1(W1ub_s, a1ub_s, W1bu_d, a1bu_d),
                       _wcat1(W1bu_s, a1bu_s, W1ub_d, a1ub_d)])

    su, du, eu = _pad_edges(edge_index_u2b, edge_weight_u2b)
    sb, db, eb = _pad_edges(edge_index_b2u, edge_weight_b2u)
    src_st = jnp.stack([su, sb])
    dst_st = jnp.stack([du, db])
    ew_st = jnp.stack([eu, eb])

    bias1 = jnp.stack([b1ub, b1bu])
    bias2 = jnp.stack([jnp.tile(b2ub, 8), jnp.tile(b2bu, 8)])

    h1, s1, d1 = _tc1(x_st, w1cat)
    agg1 = _make_sc(16)(h1, s1, d1, src_st, dst_st, ew_st, bias1)

    def _wcat2(ws, a_s, wd_other, a_d_other):
        return jnp.concatenate(
            [ws, (ws @ a_s)[:, None], (wd_other @ a_d_other)[:, None],
             jnp.zeros((16, 4), _F32)], axis=1)

    w2cat = jnp.stack([_wcat2(W2ub_s, a2ub_s, W2bu_d, a2bu_d),
                       _wcat2(W2bu_s, a2bu_s, W2ub_d, a2ub_d)])

    h2, s2, d2 = _tc2(agg1, w2cat)
    h2f = h2.reshape(2, NPAD * 2)
    agg2 = _make_sc(2)(h2f, s2, d2, src_st, dst_st, ew_st, bias2)

    badge2 = agg2[0].reshape(NPAD, 2)[:N]
    user2 = agg2[1].reshape(NPAD, 2)[:N]
    return (user2, badge2)


# speedup vs baseline: 70.6006x; 1.0672x over previous
import functools

import jax
import jax.numpy as jnp
from jax import lax
from jax.experimental import pallas as pl
from jax.experimental.pallas import tpu as pltpu
from jax.experimental.pallas import tpu_sc as plsc

N = 10000
D = 128
E = 320000
NPAD = 10240
RPT = NPAD // 16
NTILE = 16
CH = 157
EPT = CH * 128
EP = NTILE * EPT

_F32 = jnp.float32
_I32 = jnp.int32


def _tc1_body(x_ref, w_ref, h_ref, s_ref, d_ref):
    y = jnp.dot(x_ref[0], w_ref[0], preferred_element_type=_F32)
    h_ref[0] = y[:, :16]
    s_ref[0, 0] = y[:, 16]
    d_ref[0, 0] = y[:, 17]


_tc1 = pl.pallas_call(
    _tc1_body,
    grid=(2,),
    in_specs=[
        pl.BlockSpec((1, NPAD, D), lambda t: (t, 0, 0)),
        pl.BlockSpec((1, D, 24), lambda t: (t, 0, 0)),
    ],
    out_specs=[
        pl.BlockSpec((1, NPAD, 16), lambda t: (t, 0, 0)),
        pl.BlockSpec((1, 1, NPAD), lambda t: (t, 0, 0)),
        pl.BlockSpec((1, 1, NPAD), lambda t: (t, 0, 0)),
    ],
    out_shape=[
        jax.ShapeDtypeStruct((2, NPAD, 16), _F32),
        jax.ShapeDtypeStruct((2, 1, NPAD), _F32),
        jax.ShapeDtypeStruct((2, 1, NPAD), _F32),
    ],
)


def _tc2_body(agg_ref, w_ref, h_ref, s_ref, d_ref):
    x2 = jnp.maximum(agg_ref[0], 0.0)
    y = jnp.dot(x2, w_ref[0], preferred_element_type=_F32)
    h_ref[0] = y[:, :2]
    s_ref[0, 0] = y[:, 2]
    d_ref[0, 0] = y[:, 3]


_tc2 = pl.pallas_call(
    _tc2_body,
    grid=(2,),
    in_specs=[
        pl.BlockSpec((1, NPAD, 16), lambda t: (1 - t, 0, 0)),
        pl.BlockSpec((1, 16, 8), lambda t: (t, 0, 0)),
    ],
    out_specs=[
        pl.BlockSpec((1, NPAD, 2), lambda t: (t, 0, 0)),
        pl.BlockSpec((1, 1, NPAD), lambda t: (t, 0, 0)),
        pl.BlockSpec((1, 1, NPAD), lambda t: (t, 0, 0)),
    ],
    out_shape=[
        jax.ShapeDtypeStruct((2, NPAD, 2), _F32),
        jax.ShapeDtypeStruct((2, 1, NPAD), _F32),
        jax.ShapeDtypeStruct((2, 1, NPAD), _F32),
    ],
)


@functools.lru_cache(maxsize=None)
def _make_sc(h_dim):
    mesh = plsc.VectorSubcoreMesh(core_axis_name="c", subcore_axis_name="s",
                                  num_cores=2, num_subcores=NTILE)
    if h_dim == 16:
        out_type = jax.ShapeDtypeStruct((2, NPAD, 16), _F32)
        scratch = [
            pltpu.VMEM((CH, 128), _I32),
            pltpu.VMEM((CH, 128), _I32),
            pltpu.VMEM((128,), _F32),
            pltpu.VMEM((CH, 128), _F32),
            pltpu.VMEM((NPAD,), _F32),
            pltpu.VMEM((NPAD,), _F32),
            pltpu.VMEM((RPT, 16), _F32),
            pltpu.VMEM((128, 16), _F32),
            pltpu.VMEM((16,), _F32),
            pltpu.VMEM((16, 16), _F32),
            pltpu.VMEM((16,), _F32),
            pltpu.VMEM((RPT,), _F32),
            pltpu.SemaphoreType.DMA,
            pltpu.MemorySpace.VMEM_SHARED((NPAD, 16), _F32),
            pltpu.MemorySpace.VMEM_SHARED((NPAD, 16), _F32),
            pltpu.MemorySpace.VMEM_SHARED((NPAD,), _F32),
            pltpu.MemorySpace.VMEM_SHARED((16, 16), _F32),
        ]
    else:
        out_type = jax.ShapeDtypeStruct((2, NPAD * 2), _F32)
        scratch = [
            pltpu.VMEM((CH, 128), _I32),
            pltpu.VMEM((CH, 128), _I32),
            pltpu.VMEM((128,), _F32),
            pltpu.VMEM((CH, 128), _F32),
            pltpu.VMEM((NPAD,), _F32),
            pltpu.VMEM((NPAD,), _F32),
            pltpu.VMEM((NPAD * 2,), _F32),
            pltpu.VMEM((2, 128), _F32),
            pltpu.VMEM((2, 128), _I32),
            pltpu.VMEM((2 * RPT,), _F32),
            pltpu.VMEM((16,), _F32),
            pltpu.VMEM((16, 16), _F32),
            pltpu.VMEM((16,), _F32),
            pltpu.VMEM((RPT,), _F32),
            pltpu.SemaphoreType.DMA,
            pltpu.MemorySpace.VMEM_SHARED((NPAD * 2,), _F32),
            pltpu.MemorySpace.VMEM_SHARED((NPAD,), _F32),
            pltpu.MemorySpace.VMEM_SHARED((16, 16), _F32),
        ]

    def body(h_hbm, s_hbm, d_hbm, src_hbm, dst_hbm, ew_hbm, b_hbm, out_hbm,
             *scr):
        if h_dim == 16:
            (srcv, dstv, ewrow, work, sbuf, dbuf, hstage, rows, maxv, maxall,
             bbuf, zv, densem, h_sp, out_sp, den_sp, max_sp) = scr
        else:
            (srcv, dstv, ewrow, work, sbuf, dbuf, h2d, vals, dstx2, ostage,
             maxv, maxall, bbuf, zv, densem, out_sp, den_sp, max_sp) = scr
        w = lax.axis_index("s")
        rel = lax.axis_index("c")
        r0 = w * RPT
        zero16 = jnp.zeros((16,), _F32)

        pltpu.sync_copy(src_hbm.at[rel, w], srcv)
        pltpu.sync_copy(dst_hbm.at[rel, w], dstv)
        pltpu.sync_copy(s_hbm.at[rel, 0], sbuf)
        pltpu.sync_copy(d_hbm.at[1 - rel, 0], dbuf)
        pltpu.sync_copy(b_hbm.at[rel], bbuf)

        def _zv_row(i, _):
            zv[pl.ds(i * 16, 16)] = zero16
            return 0
        lax.fori_loop(0, RPT // 16, _zv_row, 0)
        pltpu.sync_copy(zv.at[pl.ds(0, RPT)], den_sp.at[pl.ds(r0, RPT)])

        if h_dim == 16:
            pltpu.sync_copy(h_hbm.at[rel, pl.ds(r0, RPT)], hstage)
            pltpu.sync_copy(hstage, h_sp.at[pl.ds(r0, RPT)])

            def _z_row(r, _):
                hstage[r, :] = zero16
                return 0
            lax.fori_loop(0, RPT, _z_row, 0)
            pltpu.sync_copy(hstage, out_sp.at[pl.ds(r0, RPT)])
        else:
            pltpu.sync_copy(h_hbm.at[rel], h2d)

            def _z_row2(i, _):
                ostage[pl.ds(i * 16, 16)] = zero16
                return 0
            lax.fori_loop(0, (2 * RPT) // 16, _z_row2, 0)
            pltpu.sync_copy(ostage, out_sp.at[pl.ds(2 * r0, 2 * RPT)])

        neg = jnp.full((16,), -3.0e38, _F32)

        def _p1_row(cc, runmax):
            rm = runmax
            for k in range(8):
                si = srcv[cc, pl.ds(k * 16, 16)]
                di = dstv[cc, pl.ds(k * 16, 16)]
                sv = plsc.load_gather(sbuf, [si])
                dv = plsc.load_gather(dbuf, [di])
                logit = sv + dv
                logit = jnp.where(logit > 0.0, logit, 0.2 * logit)
                work[cc, pl.ds(k * 16, 16)] = logit
                rm = jnp.maximum(rm, logit)
            return rm

        runmax = lax.fori_loop(0, CH, _p1_row, neg)
        maxv[:] = jnp.broadcast_to(jnp.max(runmax), (16,))
        pltpu.sync_copy(maxv, max_sp.at[w])
        plsc.subcore_barrier()

        pltpu.sync_copy(max_sp, maxall)

        def _mred(i, mm):
            return jnp.maximum(mm, maxall[i, :])
        m_glob = jnp.max(lax.fori_loop(0, 16, _mred, neg))

        def _p2_exp(cc):
            for k in range(8):
                logit = work[cc, pl.ds(k * 16, 16)]
                work[cc, pl.ds(k * 16, 16)] = jnp.exp(logit - m_glob)

        def _den_issue(cc):
            pltpu.async_copy(work.at[cc], den_sp.at[dstv.at[cc]], densem,
                             add=True)

        def _den_wait(cc):
            pltpu.make_async_copy(work.at[cc], den_sp.at[dstv.at[cc]],
                                  densem).wait()

        for cc in range(4):
            _p2_exp(cc)
            _den_issue(cc)

        def _p2_row(cc, _):
            _p2_exp(cc)
            _den_issue(cc)
            _den_wait(cc - 4)
            return 0
        lax.fori_loop(4, CH, _p2_row, 0)
        for cc in range(CH - 4, CH):
            _den_wait(cc)
        plsc.subcore_barrier()
        pltpu.sync_copy(den_sp, sbuf)

        iota = lax.iota(_I32, 16)
        zi = iota * 0
        oi = zi + 1

        def _p34_row(cc, _):
            pltpu.sync_copy(ew_hbm.at[rel, w, cc], ewrow)

            for k in range(8):
                ex = work[cc, pl.ds(k * 16, 16)]
                di = dstv[cc, pl.ds(k * 16, 16)]
                den = plsc.load_gather(sbuf, [di])
                alpha = ex / (den + 1e-16)
                work[cc, pl.ds(k * 16, 16)] = (
                    alpha * ewrow[pl.ds(k * 16, 16)])
            if h_dim == 16:
                pltpu.sync_copy(h_sp.at[srcv.at[cc]], rows)

                def _scale_e(e, __):
                    ce = plsc.load_gather(work.at[cc], [zi + e])
                    rows[e, :] = rows[e, :] * ce
                    return 0
                lax.fori_loop(0, 128, _scale_e, 0, unroll=8)
                pltpu.sync_copy(rows, out_sp.at[dstv.at[cc]], add=True)
            else:
                for k in range(8):
                    si = srcv[cc, pl.ds(k * 16, 16)]
                    di = dstv[cc, pl.ds(k * 16, 16)]
                    co = work[cc, pl.ds(k * 16, 16)]
                    si2 = si * 2
                    vals[0, pl.ds(k * 16, 16)] = \
                        plsc.load_gather(h2d, [si2]) * co
                    vals[1, pl.ds(k * 16, 16)] = \
                        plsc.load_gather(h2d, [si2 + 1]) * co
                    dstx2[0, pl.ds(k * 16, 16)] = di * 2
                    dstx2[1, pl.ds(k * 16, 16)] = di * 2 + 1
                pltpu.sync_copy(vals.at[0], out_sp.at[dstx2.at[0]], add=True)
                pltpu.sync_copy(vals.at[1], out_sp.at[dstx2.at[1]], add=True)
            return 0
        lax.fori_loop(0, CH, _p34_row, 0)
        plsc.subcore_barrier()

        bv = bbuf[:]
        if h_dim == 16:
            pltpu.sync_copy(out_sp.at[pl.ds(r0, RPT)], hstage)

            def _b_row(r, _):
                hstage[r, :] = hstage[r, :] + bv
                return 0
            lax.fori_loop(0, RPT, _b_row, 0)
            pltpu.sync_copy(hstage, out_hbm.at[rel, pl.ds(r0, RPT)])
        else:
            o0 = 2 * r0
            pltpu.sync_copy(out_sp.at[pl.ds(o0, 2 * RPT)], ostage)

            def _b_row2(i, _):
                ostage[pl.ds(i * 16, 16)] = ostage[pl.ds(i * 16, 16)] + bv
                return 0
            lax.fori_loop(0, (2 * RPT) // 16, _b_row2, 0)
            pltpu.sync_copy(ostage, out_hbm.at[rel, pl.ds(o0, 2 * RPT)])

    return pl.kernel(
        body, out_type=out_type, mesh=mesh, scratch_types=scratch,
        compiler_params=pltpu.CompilerParams(needs_layout_passes=False,
                                             use_tc_tiling_on_sc=False))


def _pad_edges(ei, ew):
    src = jnp.concatenate([ei[0], jnp.full((EP - E,), N, _I32)])
    dst = jnp.concatenate([ei[1], jnp.full((EP - E,), N, _I32)])
    ewp = jnp.concatenate([ew, jnp.zeros((EP - E,), _F32)])
    return (src.reshape(NTILE, CH, 128), dst.reshape(NTILE, CH, 128),
            ewp.reshape(NTILE, CH, 128))


def kernel(x_user, x_badge, edge_index_u2b, edge_index_b2u,
           edge_weight_u2b, edge_weight_b2u,
           W1ub_s, W1ub_d, a1ub_s, a1ub_d, b1ub,
           W1bu_s, W1bu_d, a1bu_s, a1bu_d, b1bu,
           W2ub_s, W2ub_d, a2ub_s, a2ub_d, b2ub,
           W2bu_s, W2bu_d, a2bu_s, a2bu_d, b2bu):
    pad = ((0, NPAD - N), (0, 0))
    x_st = jnp.stack([jnp.pad(x_user, pad), jnp.pad(x_badge, pad)])

    def _wcat1(ws, a_s, wd_other, a_d_other):
        return jnp.concatenate(
            [ws, (ws @ a_s)[:, None], (wd_other @ a_d_other)[:, None],
             jnp.zeros((D, 6), _F32)], axis=1)

    w1cat = jnp.stack([_wcat1(W1ub_s, a1ub_s, W1bu_d, a1bu_d),
                       _wcat1(W1bu_s, a1bu_s, W1ub_d, a1ub_d)])

    su, du, eu = _pad_edges(edge_index_u2b, edge_weight_u2b)
    sb, db, eb = _pad_edges(edge_index_b2u, edge_weight_b2u)
    src_st = jnp.stack([su, sb])
    dst_st = jnp.stack([du, db])
    ew_st = jnp.stack([eu, eb])

    bias1 = jnp.stack([b1ub, b1bu])
    bias2 = jnp.stack([jnp.tile(b2ub, 8), jnp.tile(b2bu, 8)])

    h1, s1, d1 = _tc1(x_st, w1cat)
    agg1 = _make_sc(16)(h1, s1, d1, src_st, dst_st, ew_st, bias1)

    def _wcat2(ws, a_s, wd_other, a_d_other):
        return jnp.concatenate(
            [ws, (ws @ a_s)[:, None], (wd_other @ a_d_other)[:, None],
             jnp.zeros((16, 4), _F32)], axis=1)

    w2cat = jnp.stack([_wcat2(W2ub_s, a2ub_s, W2bu_d, a2bu_d),
                       _wcat2(W2bu_s, a2bu_s, W2ub_d, a2ub_d)])

    h2, s2, d2 = _tc2(agg1, w2cat)
    h2f = h2.reshape(2, NPAD * 2)
    agg2 = _make_sc(2)(h2f, s2, d2, src_st, dst_st, ew_st, bias2)

    badge2 = agg2[0].reshape(NPAD, 2)[:N]
    user2 = agg2[1].reshape(NPAD, 2)[:N]
    return (user2, badge2)


# speedup vs baseline: 93.5772x; 1.3254x over previous
import functools

import jax
import jax.numpy as jnp
from jax import lax
from jax.experimental import pallas as pl
from jax.experimental.pallas import tpu as pltpu
from jax.experimental.pallas import tpu_sc as plsc

N = 10000
D = 128
E = 320000
NPAD = 10240
RPT = NPAD // 16
NTILE = 16
CH = 157
EPT = CH * 128
EP = NTILE * EPT

_F32 = jnp.float32
_I32 = jnp.int32


def _tc1_body(x_ref, w_ref, h_ref, s_ref, d_ref):
    y = jnp.dot(x_ref[0], w_ref[0], preferred_element_type=_F32)
    h_ref[0] = y[:, :16]
    s_ref[0, 0] = y[:, 16]
    d_ref[0, 0] = y[:, 17]


_tc1 = pl.pallas_call(
    _tc1_body,
    grid=(2,),
    in_specs=[
        pl.BlockSpec((1, NPAD, D), lambda t: (t, 0, 0)),
        pl.BlockSpec((1, D, 24), lambda t: (t, 0, 0)),
    ],
    out_specs=[
        pl.BlockSpec((1, NPAD, 16), lambda t: (t, 0, 0)),
        pl.BlockSpec((1, 1, NPAD), lambda t: (t, 0, 0)),
        pl.BlockSpec((1, 1, NPAD), lambda t: (t, 0, 0)),
    ],
    out_shape=[
        jax.ShapeDtypeStruct((2, NPAD, 16), _F32),
        jax.ShapeDtypeStruct((2, 1, NPAD), _F32),
        jax.ShapeDtypeStruct((2, 1, NPAD), _F32),
    ],
)


def _tc2_body(agg_ref, w_ref, h_ref, s_ref, d_ref):
    x2 = jnp.maximum(agg_ref[0], 0.0)
    y = jnp.dot(x2, w_ref[0], preferred_element_type=_F32)
    h_ref[0] = y[:, :2]
    s_ref[0, 0] = y[:, 2]
    d_ref[0, 0] = y[:, 3]


_tc2 = pl.pallas_call(
    _tc2_body,
    grid=(2,),
    in_specs=[
        pl.BlockSpec((1, NPAD, 16), lambda t: (1 - t, 0, 0)),
        pl.BlockSpec((1, 16, 8), lambda t: (t, 0, 0)),
    ],
    out_specs=[
        pl.BlockSpec((1, NPAD, 2), lambda t: (t, 0, 0)),
        pl.BlockSpec((1, 1, NPAD), lambda t: (t, 0, 0)),
        pl.BlockSpec((1, 1, NPAD), lambda t: (t, 0, 0)),
    ],
    out_shape=[
        jax.ShapeDtypeStruct((2, NPAD, 2), _F32),
        jax.ShapeDtypeStruct((2, 1, NPAD), _F32),
        jax.ShapeDtypeStruct((2, 1, NPAD), _F32),
    ],
)


@functools.lru_cache(maxsize=None)
def _make_sc(h_dim):
    mesh = plsc.VectorSubcoreMesh(core_axis_name="c", subcore_axis_name="s",
                                  num_cores=2, num_subcores=NTILE)
    if h_dim == 16:
        out_type = jax.ShapeDtypeStruct((2, NPAD, 16), _F32)
        scratch = [
            pltpu.VMEM((CH, 128), _I32),
            pltpu.VMEM((CH, 128), _I32),
            pltpu.VMEM((4, 128), _F32),
            pltpu.VMEM((CH, 128), _F32),
            pltpu.VMEM((NPAD,), _F32),
            pltpu.VMEM((NPAD,), _F32),
            pltpu.VMEM((RPT, 16), _F32),
            pltpu.VMEM((4, 128, 16), _F32),
            pltpu.VMEM((16,), _F32),
            pltpu.VMEM((16, 16), _F32),
            pltpu.VMEM((16,), _F32),
            pltpu.VMEM((RPT,), _F32),
            pltpu.SemaphoreType.DMA,
            pltpu.SemaphoreType.DMA((4,)),
            pltpu.SemaphoreType.DMA((4,)),
            pltpu.SemaphoreType.DMA((4,)),
            pltpu.MemorySpace.VMEM_SHARED((NPAD, 16), _F32),
            pltpu.MemorySpace.VMEM_SHARED((NPAD, 16), _F32),
            pltpu.MemorySpace.VMEM_SHARED((NPAD,), _F32),
            pltpu.MemorySpace.VMEM_SHARED((16, 16), _F32),
        ]
    else:
        out_type = jax.ShapeDtypeStruct((2, NPAD * 2), _F32)
        scratch = [
            pltpu.VMEM((CH, 128), _I32),
            pltpu.VMEM((CH, 128), _I32),
            pltpu.VMEM((2, 128), _F32),
            pltpu.VMEM((CH, 128), _F32),
            pltpu.VMEM((NPAD,), _F32),
            pltpu.VMEM((NPAD,), _F32),
            pltpu.VMEM((NPAD * 2,), _F32),
            pltpu.VMEM((2, 2, 128), _F32),
            pltpu.VMEM((2, 2, 128), _I32),
            pltpu.VMEM((2 * RPT,), _F32),
            pltpu.VMEM((16,), _F32),
            pltpu.VMEM((16, 16), _F32),
            pltpu.VMEM((16,), _F32),
            pltpu.VMEM((RPT,), _F32),
            pltpu.SemaphoreType.DMA,
            pltpu.SemaphoreType.DMA((2,)),
            pltpu.SemaphoreType.DMA((2,)),
            pltpu.SemaphoreType.DMA((2,)),
            pltpu.MemorySpace.VMEM_SHARED((NPAD * 2,), _F32),
            pltpu.MemorySpace.VMEM_SHARED((NPAD,), _F32),
            pltpu.MemorySpace.VMEM_SHARED((16, 16), _F32),
        ]

    def body(h_hbm, s_hbm, d_hbm, src_hbm, dst_hbm, ew_hbm, b_hbm, out_hbm,
             *scr):
        if h_dim == 16:
            (srcv, dstv, ewrow, work, sbuf, dbuf, hstage, rows, maxv, maxall,
             bbuf, zv, densem, gsem, ssem, esem, h_sp, out_sp, den_sp,
             max_sp) = scr
        else:
            (srcv, dstv, ewrow, work, sbuf, dbuf, h2d, vals, dstx2, ostage,
             maxv, maxall, bbuf, zv, densem, gsem, ssem, esem, out_sp,
             den_sp, max_sp) = scr
        w = lax.axis_index("s")
        rel = lax.axis_index("c")
        r0 = w * RPT
        zero16 = jnp.zeros((16,), _F32)

        pltpu.sync_copy(src_hbm.at[rel, w], srcv)
        pltpu.sync_copy(dst_hbm.at[rel, w], dstv)
        pltpu.sync_copy(s_hbm.at[rel, 0], sbuf)
        pltpu.sync_copy(d_hbm.at[1 - rel, 0], dbuf)
        pltpu.sync_copy(b_hbm.at[rel], bbuf)

        def _zv_row(i, _):
            zv[pl.ds(i * 16, 16)] = zero16
            return 0
        lax.fori_loop(0, RPT // 16, _zv_row, 0)
        pltpu.sync_copy(zv.at[pl.ds(0, RPT)], den_sp.at[pl.ds(r0, RPT)])

        if h_dim == 16:
            pltpu.sync_copy(h_hbm.at[rel, pl.ds(r0, RPT)], hstage)
            pltpu.sync_copy(hstage, h_sp.at[pl.ds(r0, RPT)])

            def _z_row(r, _):
                hstage[r, :] = zero16
                return 0
            lax.fori_loop(0, RPT, _z_row, 0)
            pltpu.sync_copy(hstage, out_sp.at[pl.ds(r0, RPT)])
        else:
            pltpu.sync_copy(h_hbm.at[rel], h2d)

            def _z_row2(i, _):
                ostage[pl.ds(i * 16, 16)] = zero16
                return 0
            lax.fori_loop(0, (2 * RPT) // 16, _z_row2, 0)
            pltpu.sync_copy(ostage, out_sp.at[pl.ds(2 * r0, 2 * RPT)])

        neg = jnp.full((16,), -3.0e38, _F32)

        def _p1_row(cc, runmax):
            rm = runmax
            for k in range(8):
                si = srcv[cc, pl.ds(k * 16, 16)]
                di = dstv[cc, pl.ds(k * 16, 16)]
                sv = plsc.load_gather(sbuf, [si])
                dv = plsc.load_gather(dbuf, [di])
                logit = sv + dv
                logit = jnp.where(logit > 0.0, logit, 0.2 * logit)
                work[cc, pl.ds(k * 16, 16)] = logit
                rm = jnp.maximum(rm, logit)
            return rm

        runmax = lax.fori_loop(0, CH, _p1_row, neg)
        maxv[:] = jnp.broadcast_to(jnp.max(runmax), (16,))
        pltpu.sync_copy(maxv, max_sp.at[w])
        plsc.subcore_barrier()

        pltpu.sync_copy(max_sp, maxall)

        def _mred(i, mm):
            return jnp.maximum(mm, maxall[i, :])
        m_glob = jnp.max(lax.fori_loop(0, 16, _mred, neg))

        def _p2_exp(cc):
            for k in range(8):
                logit = work[cc, pl.ds(k * 16, 16)]
                work[cc, pl.ds(k * 16, 16)] = jnp.exp(logit - m_glob)

        def _den_issue(cc):
            pltpu.async_copy(work.at[cc], den_sp.at[dstv.at[cc]], densem,
                             add=True)

        def _den_wait(cc):
            pltpu.make_async_copy(work.at[cc], den_sp.at[dstv.at[cc]],
                                  densem).wait()

        for cc in range(4):
            _p2_exp(cc)
            _den_issue(cc)

        def _p2_row(cc, _):
            _p2_exp(cc)
            _den_issue(cc)
            _den_wait(cc - 4)
            return 0
        lax.fori_loop(4, CH, _p2_row, 0)
        for cc in range(CH - 4, CH):
            _den_wait(cc)
        plsc.subcore_barrier()
        pltpu.sync_copy(den_sp, sbuf)

        iota = lax.iota(_I32, 16)
        zi = iota * 0

        def _coeff_row(cc, b):
            for k in range(8):
                ex = work[cc, pl.ds(k * 16, 16)]
                di = dstv[cc, pl.ds(k * 16, 16)]
                den = plsc.load_gather(sbuf, [di])
                alpha = ex / (den + 1e-16)
                work[cc, pl.ds(k * 16, 16)] = (
                    alpha * ewrow[b, pl.ds(k * 16, 16)])

        if h_dim == 16:
            def _g_issue(cc, b):
                pltpu.async_copy(h_sp.at[srcv.at[cc]], rows.at[b],
                                 gsem.at[b])
                pltpu.async_copy(ew_hbm.at[rel, w, cc], ewrow.at[b],
                                 esem.at[b])

            def _g_wait(cc, b):
                pltpu.make_async_copy(h_sp.at[srcv.at[cc]], rows.at[b],
                                      gsem.at[b]).wait()
                pltpu.make_async_copy(ew_hbm.at[rel, w, cc], ewrow.at[b],
                                      esem.at[b]).wait()

            def _s_issue(cc, b):
                pltpu.async_copy(rows.at[b], out_sp.at[dstv.at[cc]],
                                 ssem.at[b], add=True)

            def _s_wait(cc, b):
                pltpu.make_async_copy(rows.at[b], out_sp.at[dstv.at[cc]],
                                      ssem.at[b]).wait()

            def _step(cc, b, cc2, b2, do_swait):
                _g_wait(cc, b)
                _coeff_row(cc, b)

                def _scale_e(e, __):
                    ce = plsc.load_gather(work.at[cc], [zi + e])
                    rows[b, e, :] = rows[b, e, :] * ce
                    return 0
                lax.fori_loop(0, 128, _scale_e, 0, unroll=8)
                _s_issue(cc, b)
                if cc2 is not None:
                    if do_swait:
                        _s_wait(cc2 - 4, b2)
                    _g_issue(cc2, b2)

            _g_issue(0, 0)
            _g_issue(1, 1)
            _step(0, 0, 2, 2, False)
            _step(1, 1, 3, 3, False)

            def _p4_round(r, _):
                c0 = 4 * r + 2
                for j in range(4):
                    _step(c0 + j, (2 + j) % 4, c0 + j + 2, j, True)
                return 0
            lax.fori_loop(0, 38, _p4_round, 0)
            _step(154, 2, 156, 0, True)
            _step(155, 3, None, None, False)
            _step(156, 0, None, None, False)
            _s_wait(153, 1)
            _s_wait(154, 2)
            _s_wait(155, 3)
            _s_wait(156, 0)
        else:
            def _g_issue2(cc, b):
                pltpu.async_copy(ew_hbm.at[rel, w, cc], ewrow.at[b],
                                 gsem.at[b])

            def _g_wait2(cc, b):
                pltpu.make_async_copy(ew_hbm.at[rel, w, cc], ewrow.at[b],
                                      gsem.at[b]).wait()

            def _s_issue2(cc, b):
                pltpu.async_copy(vals.at[b, 0], out_sp.at[dstx2.at[b, 0]],
                                 ssem.at[b], add=True)
                pltpu.async_copy(vals.at[b, 1], out_sp.at[dstx2.at[b, 1]],
                                 ssem.at[b], add=True)

            def _s_wait2(cc, b):
                pltpu.make_async_copy(vals.at[b, 0],
                                      out_sp.at[dstx2.at[b, 0]],
                                      ssem.at[b]).wait()
                pltpu.make_async_copy(vals.at[b, 1],
                                      out_sp.at[dstx2.at[b, 1]],
                                      ssem.at[b]).wait()

            def _step2(cc, b, cc2, do_swait):
                _g_wait2(cc, b)
                _coeff_row(cc, b)
                for k in range(8):
                    si = srcv[cc, pl.ds(k * 16, 16)]
                    di = dstv[cc, pl.ds(k * 16, 16)]
                    co = work[cc, pl.ds(k * 16, 16)]
                    si2 = si * 2
                    vals[b, 0, pl.ds(k * 16, 16)] = \
                        plsc.load_gather(h2d, [si2]) * co
                    vals[b, 1, pl.ds(k * 16, 16)] = \
                        plsc.load_gather(h2d, [si2 + 1]) * co
                    dstx2[b, 0, pl.ds(k * 16, 16)] = di * 2
                    dstx2[b, 1, pl.ds(k * 16, 16)] = di * 2 + 1
                _s_issue2(cc, b)
                if cc2 is not None:
                    if do_swait:
                        _s_wait2(cc2 - 2, 1 - b)
                    _g_issue2(cc2, 1 - b)

            _g_issue2(0, 0)
            _step2(0, 0, 1, False)

            def _p4_round2(r, _):
                c0 = 2 * r + 1
                _step2(c0, 1, c0 + 1, True)
                _step2(c0 + 1, 0, c0 + 2, True)
                return 0
            lax.fori_loop(0, 77, _p4_round2, 0)
            _step2(155, 1, 156, True)
            _step2(156, 0, None, False)
            _s_wait2(155, 1)
            _s_wait2(156, 0)
        plsc.subcore_barrier()

        bv = bbuf[:]
        if h_dim == 16:
            pltpu.sync_copy(out_sp.at[pl.ds(r0, RPT)], hstage)

            def _b_row(r, _):
                hstage[r, :] = hstage[r, :] + bv
                return 0
            lax.fori_loop(0, RPT, _b_row, 0)
            pltpu.sync_copy(hstage, out_hbm.at[rel, pl.ds(r0, RPT)])
        else:
            o0 = 2 * r0
            pltpu.sync_copy(out_sp.at[pl.ds(o0, 2 * RPT)], ostage)

            def _b_row2(i, _):
                ostage[pl.ds(i * 16, 16)] = ostage[pl.ds(i * 16, 16)] + bv
                return 0
            lax.fori_loop(0, (2 * RPT) // 16, _b_row2, 0)
            pltpu.sync_copy(ostage, out_hbm.at[rel, pl.ds(o0, 2 * RPT)])

    return pl.kernel(
        body, out_type=out_type, mesh=mesh, scratch_types=scratch,
        compiler_params=pltpu.CompilerParams(needs_layout_passes=False,
                                             use_tc_tiling_on_sc=False))


def _pad_edges(ei, ew):
    src = jnp.concatenate([ei[0], jnp.full((EP - E,), N, _I32)])
    dst = jnp.concatenate([ei[1], jnp.full((EP - E,), N, _I32)])
    ewp = jnp.concatenate([ew, jnp.zeros((EP - E,), _F32)])
    return (src.reshape(NTILE, CH, 128), dst.reshape(NTILE, CH, 128),
            ewp.reshape(NTILE, CH, 128))


def kernel(x_user, x_badge, edge_index_u2b, edge_index_b2u,
           edge_weight_u2b, edge_weight_b2u,
           W1ub_s, W1ub_d, a1ub_s, a1ub_d, b1ub,
           W1bu_s, W1bu_d, a1bu_s, a1bu_d, b1bu,
           W2ub_s, W2ub_d, a2ub_s, a2ub_d, b2ub,
           W2bu_s, W2bu_d, a2bu_s, a2bu_d, b2bu):
    pad = ((0, NPAD - N), (0, 0))
    x_st = jnp.stack([jnp.pad(x_user, pad), jnp.pad(x_badge, pad)])

    def _wcat1(ws, a_s, wd_other, a_d_other):
        return jnp.concatenate(
            [ws, (ws @ a_s)[:, None], (wd_other @ a_d_other)[:, None],
             jnp.zeros((D, 6), _F32)], axis=1)

    w1cat = jnp.stack([_wcat1(W1ub_s, a1ub_s, W1bu_d, a1bu_d),
                       _wcat1(W1bu_s, a1bu_s, W1ub_d, a1ub_d)])

    su, du, eu = _pad_edges(edge_index_u2b, edge_weight_u2b)
    sb, db, eb = _pad_edges(edge_index_b2u, edge_weight_b2u)
    src_st = jnp.stack([su, sb])
    dst_st = jnp.stack([du, db])
    ew_st = jnp.stack([eu, eb])

    bias1 = jnp.stack([b1ub, b1bu])
    bias2 = jnp.stack([jnp.tile(b2ub, 8), jnp.tile(b2bu, 8)])

    h1, s1, d1 = _tc1(x_st, w1cat)
    agg1 = _make_sc(16)(h1, s1, d1, src_st, dst_st, ew_st, bias1)

    def _wcat2(ws, a_s, wd_other, a_d_other):
        return jnp.concatenate(
            [ws, (ws @ a_s)[:, None], (wd_other @ a_d_other)[:, None],
             jnp.zeros((16, 4), _F32)], axis=1)

    w2cat = jnp.stack([_wcat2(W2ub_s, a2ub_s, W2bu_d, a2bu_d),
                       _wcat2(W2bu_s, a2bu_s, W2ub_d, a2ub_d)])

    h2, s2, d2 = _tc2(agg1, w2cat)
    h2f = h2.reshape(2, NPAD * 2)
    agg2 = _make_sc(2)(h2f, s2, d2, src_st, dst_st, ew_st, bias2)

    badge2 = agg2[0].reshape(NPAD, 2)[:N]
    user2 = agg2[1].reshape(NPAD, 2)[:N]
    return (user2, badge2)


# speedup vs baseline: 97.3674x; 1.0405x over previous
import functools

import jax
import jax.numpy as jnp
from jax import lax
from jax.experimental import pallas as pl
from jax.experimental.pallas import tpu as pltpu
from jax.experimental.pallas import tpu_sc as plsc

N = 10000
D = 128
E = 320000
NPAD = 10240
RPT = NPAD // 16
NTILE = 16
CH = 157
EPT = CH * 128
EP = NTILE * EPT

_F32 = jnp.float32
_I32 = jnp.int32


def _tc1_body(xu_ref, xb_ref, w_ref, h_ref, s_ref, d_ref):
    zpad_h = jnp.zeros((NPAD - N, 16), _F32)
    zpad_1 = jnp.zeros((NPAD - N,), _F32)
    for t, x in ((0, xu_ref[...]), (1, xb_ref[...])):
        y = jnp.dot(x, w_ref[t], preferred_element_type=_F32)
        h_ref[t, :N] = y[:, :16]
        h_ref[t, N:] = zpad_h
        s_ref[t, 0, :N] = y[:, 16]
        s_ref[t, 0, N:] = zpad_1
        d_ref[t, 0, :N] = y[:, 17]
        d_ref[t, 0, N:] = zpad_1


_tc1 = pl.pallas_call(
    _tc1_body,
    out_shape=[
        jax.ShapeDtypeStruct((2, NPAD, 16), _F32),
        jax.ShapeDtypeStruct((2, 1, NPAD), _F32),
        jax.ShapeDtypeStruct((2, 1, NPAD), _F32),
    ],
)


def _tc2_body(agg_ref, w_ref, h_ref, s_ref, d_ref):
    x2 = jnp.maximum(agg_ref[0], 0.0)
    y = jnp.dot(x2, w_ref[0], preferred_element_type=_F32)
    h_ref[0] = y[:, :2]
    s_ref[0, 0] = y[:, 2]
    d_ref[0, 0] = y[:, 3]


_tc2 = pl.pallas_call(
    _tc2_body,
    grid=(2,),
    in_specs=[
        pl.BlockSpec((1, NPAD, 16), lambda t: (1 - t, 0, 0)),
        pl.BlockSpec((1, 16, 8), lambda t: (t, 0, 0)),
    ],
    out_specs=[
        pl.BlockSpec((1, NPAD, 2), lambda t: (t, 0, 0)),
        pl.BlockSpec((1, 1, NPAD), lambda t: (t, 0, 0)),
        pl.BlockSpec((1, 1, NPAD), lambda t: (t, 0, 0)),
    ],
    out_shape=[
        jax.ShapeDtypeStruct((2, NPAD, 2), _F32),
        jax.ShapeDtypeStruct((2, 1, NPAD), _F32),
        jax.ShapeDtypeStruct((2, 1, NPAD), _F32),
    ],
)


@functools.lru_cache(maxsize=None)
def _make_sc(h_dim):
    mesh = plsc.VectorSubcoreMesh(core_axis_name="c", subcore_axis_name="s",
                                  num_cores=2, num_subcores=NTILE)
    if h_dim == 16:
        out_type = jax.ShapeDtypeStruct((2, NPAD, 16), _F32)
        scratch = [
            pltpu.VMEM((CH, 128), _I32),
            pltpu.VMEM((CH, 128), _I32),
            pltpu.VMEM((4, 128), _F32),
            pltpu.VMEM((CH, 128), _F32),
            pltpu.VMEM((NPAD,), _F32),
            pltpu.VMEM((NPAD,), _F32),
            pltpu.VMEM((RPT, 16), _F32),
            pltpu.VMEM((4, 128, 16), _F32),
            pltpu.VMEM((16,), _F32),
            pltpu.VMEM((16, 16), _F32),
            pltpu.VMEM((16,), _F32),
            pltpu.VMEM((RPT,), _F32),
            pltpu.SemaphoreType.DMA,
            pltpu.SemaphoreType.DMA,
            pltpu.SemaphoreType.DMA((4,)),
            pltpu.SemaphoreType.DMA((4,)),
            pltpu.SemaphoreType.DMA((4,)),
            pltpu.MemorySpace.VMEM_SHARED((NPAD, 16), _F32),
            pltpu.MemorySpace.VMEM_SHARED((NPAD, 16), _F32),
            pltpu.MemorySpace.VMEM_SHARED((NPAD,), _F32),
            pltpu.MemorySpace.VMEM_SHARED((16, 16), _F32),
        ]
    else:
        out_type = jax.ShapeDtypeStruct((2, NPAD * 2), _F32)
        scratch = [
            pltpu.VMEM((CH, 128), _I32),
            pltpu.VMEM((CH, 128), _I32),
            pltpu.VMEM((2, 128), _F32),
            pltpu.VMEM((CH, 128), _F32),
            pltpu.VMEM((NPAD,), _F32),
            pltpu.VMEM((NPAD,), _F32),
            pltpu.VMEM((NPAD * 2,), _F32),
            pltpu.VMEM((2, 2, 128), _F32),
            pltpu.VMEM((2, 2, 128), _I32),
            pltpu.VMEM((2 * RPT,), _F32),
            pltpu.VMEM((16,), _F32),
            pltpu.VMEM((16, 16), _F32),
            pltpu.VMEM((16,), _F32),
            pltpu.VMEM((RPT,), _F32),
            pltpu.SemaphoreType.DMA,
            pltpu.SemaphoreType.DMA,
            pltpu.SemaphoreType.DMA((2,)),
            pltpu.SemaphoreType.DMA((2,)),
            pltpu.SemaphoreType.DMA((2,)),
            pltpu.MemorySpace.VMEM_SHARED((NPAD * 2,), _F32),
            pltpu.MemorySpace.VMEM_SHARED((NPAD,), _F32),
            pltpu.MemorySpace.VMEM_SHARED((16, 16), _F32),
        ]

    def body(h_hbm, s_hbm, d_hbm, su_hbm, du_hbm, sb_hbm, db_hbm, ew_hbm,
             b_hbm, out_hbm, *scr):
        if h_dim == 16:
            (srcv, dstv, ewrow, work, sbuf, dbuf, hstage, rows, maxv, maxall,
             bbuf, zv, psem, densem, gsem, ssem, esem, h_sp, out_sp, den_sp,
             max_sp) = scr
        else:
            (srcv, dstv, ewrow, work, sbuf, dbuf, h2d, vals, dstx2, ostage,
             maxv, maxall, bbuf, zv, psem, densem, gsem, ssem, esem, out_sp,
             den_sp, max_sp) = scr
        w = lax.axis_index("s")
        rel = lax.axis_index("c")
        r0 = w * RPT
        zero16 = jnp.zeros((16,), _F32)

        @pl.when(rel == 0)
        def _():
            pltpu.async_copy(su_hbm.at[w], srcv, psem)
            pltpu.async_copy(du_hbm.at[w], dstv, psem)

        @pl.when(rel == 1)
        def _():
            pltpu.async_copy(sb_hbm.at[w], srcv, psem)
            pltpu.async_copy(db_hbm.at[w], dstv, psem)
        pltpu.async_copy(s_hbm.at[rel, 0], sbuf, psem)
        pltpu.async_copy(d_hbm.at[1 - rel, 0], dbuf, psem)
        pltpu.async_copy(b_hbm.at[rel], bbuf, psem)

        if h_dim == 16:
            pltpu.async_copy(h_hbm.at[rel, pl.ds(r0, RPT)], hstage, psem)
        else:
            pltpu.async_copy(h_hbm.at[rel], h2d, psem)

        def _zv_row2(i, _):
            zv[pl.ds(i * 16, 16)] = zero16
            return 0
        lax.fori_loop(0, RPT // 16, _zv_row2, 0)
        pltpu.sync_copy(zv.at[pl.ds(0, RPT)], den_sp.at[pl.ds(r0, RPT)])

        @pl.when(rel == 0)
        def _():
            pltpu.make_async_copy(su_hbm.at[w], srcv, psem).wait()
            pltpu.make_async_copy(du_hbm.at[w], dstv, psem).wait()

        @pl.when(rel == 1)
        def _():
            pltpu.make_async_copy(sb_hbm.at[w], srcv, psem).wait()
            pltpu.make_async_copy(db_hbm.at[w], dstv, psem).wait()
        pltpu.make_async_copy(s_hbm.at[rel, 0], sbuf, psem).wait()
        pltpu.make_async_copy(d_hbm.at[1 - rel, 0], dbuf, psem).wait()
        pltpu.make_async_copy(b_hbm.at[rel], bbuf, psem).wait()

        if h_dim == 16:
            pltpu.make_async_copy(h_hbm.at[rel, pl.ds(r0, RPT)], hstage,
                                  psem).wait()
            pltpu.sync_copy(hstage, h_sp.at[pl.ds(r0, RPT)])

            def _z_row(r, _):
                hstage[r, :] = zero16
                return 0
            lax.fori_loop(0, RPT, _z_row, 0)
            pltpu.sync_copy(hstage, out_sp.at[pl.ds(r0, RPT)])
        else:
            pltpu.make_async_copy(h_hbm.at[rel], h2d, psem).wait()

            def _z_row2(i, _):
                ostage[pl.ds(i * 16, 16)] = zero16
                return 0
            lax.fori_loop(0, (2 * RPT) // 16, _z_row2, 0)
            pltpu.sync_copy(ostage, out_sp.at[pl.ds(2 * r0, 2 * RPT)])

        neg = jnp.full((16,), -3.0e38, _F32)

        def _p1_row(cc, runmax):
            rm = runmax
            for k in range(8):
                si = srcv[cc, pl.ds(k * 16, 16)]
                di = dstv[cc, pl.ds(k * 16, 16)]
                sv = plsc.load_gather(sbuf, [si])
                dv = plsc.load_gather(dbuf, [di])
                logit = sv + dv
                logit = jnp.where(logit > 0.0, logit, 0.2 * logit)
                work[cc, pl.ds(k * 16, 16)] = logit
                rm = jnp.maximum(rm, logit)
            return rm

        runmax = lax.fori_loop(0, CH, _p1_row, neg)
        maxv[:] = jnp.broadcast_to(jnp.max(runmax), (16,))
        pltpu.sync_copy(maxv, max_sp.at[w])
        plsc.subcore_barrier()

        pltpu.sync_copy(max_sp, maxall)

        def _mred(i, mm):
            return jnp.maximum(mm, maxall[i, :])
        m_glob = jnp.max(lax.fori_loop(0, 16, _mred, neg))

        def _p2_exp(cc):
            for k in range(8):
                logit = work[cc, pl.ds(k * 16, 16)]
                work[cc, pl.ds(k * 16, 16)] = jnp.exp(logit - m_glob)

        def _den_issue(cc):
            pltpu.async_copy(work.at[cc], den_sp.at[dstv.at[cc]], densem,
                             add=True)

        def _den_wait(cc):
            pltpu.make_async_copy(work.at[cc], den_sp.at[dstv.at[cc]],
                                  densem).wait()

        for cc in range(4):
            _p2_exp(cc)
            _den_issue(cc)

        def _p2_row(cc, _):
            _p2_exp(cc)
            _den_issue(cc)
            _den_wait(cc - 4)
            return 0
        lax.fori_loop(4, CH, _p2_row, 0)
        for cc in range(CH - 4, CH):
            _den_wait(cc)
        plsc.subcore_barrier()
        pltpu.sync_copy(den_sp, sbuf)

        iota = lax.iota(_I32, 16)
        zi = iota * 0

        def _coeff_row(cc, b):
            for k in range(8):
                ex = work[cc, pl.ds(k * 16, 16)]
                di = dstv[cc, pl.ds(k * 16, 16)]
                den = plsc.load_gather(sbuf, [di])
                alpha = ex / (den + 1e-16)
                work[cc, pl.ds(k * 16, 16)] = (
                    alpha * ewrow[b, pl.ds(k * 16, 16)])

        if h_dim == 16:
            def _g_issue(cc, b):
                pltpu.async_copy(h_sp.at[srcv.at[cc]], rows.at[b],
                                 gsem.at[b])
                pltpu.async_copy(ew_hbm.at[rel, w, cc], ewrow.at[b],
                                 esem.at[b])

            def _g_wait(cc, b):
                pltpu.make_async_copy(h_sp.at[srcv.at[cc]], rows.at[b],
                                      gsem.at[b]).wait()
                pltpu.make_async_copy(ew_hbm.at[rel, w, cc], ewrow.at[b],
                                      esem.at[b]).wait()

            def _s_issue(cc, b):
                pltpu.async_copy(rows.at[b], out_sp.at[dstv.at[cc]],
                                 ssem.at[b], add=True)

            def _s_wait(cc, b):
                pltpu.make_async_copy(rows.at[b], out_sp.at[dstv.at[cc]],
                                      ssem.at[b]).wait()

            def _step(cc, b, cc2, b2, do_swait):
                _g_wait(cc, b)
                _coeff_row(cc, b)

                def _scale_e(e, __):
                    ce = plsc.load_gather(work.at[cc], [zi + e])
                    rows[b, e, :] = rows[b, e, :] * ce
                    return 0
                lax.fori_loop(0, 128, _scale_e, 0, unroll=8)
                _s_issue(cc, b)
                if cc2 is not None:
                    if do_swait:
                        _s_wait(cc2 - 4, b2)
                    _g_issue(cc2, b2)

            _g_issue(0, 0)
            _g_issue(1, 1)
            _step(0, 0, 2, 2, False)
            _step(1, 1, 3, 3, False)

            def _p4_round(r, _):
                c0 = 4 * r + 2
                for j in range(4):
                    _step(c0 + j, (2 + j) % 4, c0 + j + 2, j, True)
                return 0
            lax.fori_loop(0, 38, _p4_round, 0)
            _step(154, 2, 156, 0, True)
            _step(155, 3, None, None, False)
            _step(156, 0, None, None, False)
            _s_wait(153, 1)
            _s_wait(154, 2)
            _s_wait(155, 3)
            _s_wait(156, 0)
        else:
            def _g_issue2(cc, b):
                pltpu.async_copy(ew_hbm.at[rel, w, cc], ewrow.at[b],
                                 gsem.at[b])

            def _g_wait2(cc, b):
                pltpu.make_async_copy(ew_hbm.at[rel, w, cc], ewrow.at[b],
                                      gsem.at[b]).wait()

            def _s_issue2(cc, b):
                pltpu.async_copy(vals.at[b, 0], out_sp.at[dstx2.at[b, 0]],
                                 ssem.at[b], add=True)
                pltpu.async_copy(vals.at[b, 1], out_sp.at[dstx2.at[b, 1]],
                                 ssem.at[b], add=True)

            def _s_wait2(cc, b):
                pltpu.make_async_copy(vals.at[b, 0],
                                      out_sp.at[dstx2.at[b, 0]],
                                      ssem.at[b]).wait()
                pltpu.make_async_copy(vals.at[b, 1],
                                      out_sp.at[dstx2.at[b, 1]],
                                      ssem.at[b]).wait()

            def _step2(cc, b, cc2, do_swait):
                _g_wait2(cc, b)
                _coeff_row(cc, b)
                for k in range(8):
                    si = srcv[cc, pl.ds(k * 16, 16)]
                    di = dstv[cc, pl.ds(k * 16, 16)]
                    co = work[cc, pl.ds(k * 16, 16)]
                    si2 = si * 2
                    vals[b, 0, pl.ds(k * 16, 16)] = \
                        plsc.load_gather(h2d, [si2]) * co
                    vals[b, 1, pl.ds(k * 16, 16)] = \
                        plsc.load_gather(h2d, [si2 + 1]) * co
                    dstx2[b, 0, pl.ds(k * 16, 16)] = di * 2
                    dstx2[b, 1, pl.ds(k * 16, 16)] = di * 2 + 1
                _s_issue2(cc, b)
                if cc2 is not None:
                    if do_swait:
                        _s_wait2(cc2 - 2, 1 - b)
                    _g_issue2(cc2, 1 - b)

            _g_issue2(0, 0)
            _step2(0, 0, 1, False)

            def _p4_round2(r, _):
                c0 = 2 * r + 1
                _step2(c0, 1, c0 + 1, True)
                _step2(c0 + 1, 0, c0 + 2, True)
                return 0
            lax.fori_loop(0, 77, _p4_round2, 0)
            _step2(155, 1, 156, True)
            _step2(156, 0, None, False)
            _s_wait2(155, 1)
            _s_wait2(156, 0)
        plsc.subcore_barrier()

        bv = bbuf[:]
        if h_dim == 16:
            pltpu.sync_copy(out_sp.at[pl.ds(r0, RPT)], hstage)

            def _b_row(r, _):
                hstage[r, :] = hstage[r, :] + bv
                return 0
            lax.fori_loop(0, RPT, _b_row, 0)
            pltpu.sync_copy(hstage, out_hbm.at[rel, pl.ds(r0, RPT)])
        else:
            o0 = 2 * r0
            pltpu.sync_copy(out_sp.at[pl.ds(o0, 2 * RPT)], ostage)

            def _b_row2(i, _):
                ostage[pl.ds(i * 16, 16)] = ostage[pl.ds(i * 16, 16)] + bv
                return 0
            lax.fori_loop(0, (2 * RPT) // 16, _b_row2, 0)
            pltpu.sync_copy(ostage, out_hbm.at[rel, pl.ds(o0, 2 * RPT)])

    return pl.kernel(
        body, out_type=out_type, mesh=mesh, scratch_types=scratch,
        compiler_params=pltpu.CompilerParams(needs_layout_passes=False,
                                             use_tc_tiling_on_sc=False))


def _pad_edges(ei, ew):
    src = jnp.concatenate([ei[0], jnp.full((EP - E,), N, _I32)])
    dst = jnp.concatenate([ei[1], jnp.full((EP - E,), N, _I32)])
    ewp = jnp.concatenate([ew, jnp.zeros((EP - E,), _F32)])
    return (src.reshape(NTILE, CH, 128), dst.reshape(NTILE, CH, 128),
            ewp.reshape(NTILE, CH, 128))


def kernel(x_user, x_badge, edge_index_u2b, edge_index_b2u,
           edge_weight_u2b, edge_weight_b2u,
           W1ub_s, W1ub_d, a1ub_s, a1ub_d, b1ub,
           W1bu_s, W1bu_d, a1bu_s, a1bu_d, b1bu,
           W2ub_s, W2ub_d, a2ub_s, a2ub_d, b2ub,
           W2bu_s, W2bu_d, a2bu_s, a2bu_d, b2bu):
    def _wcat1(ws, a_s, wd_other, a_d_other):
        return jnp.concatenate(
            [ws, (ws @ a_s)[:, None], (wd_other @ a_d_other)[:, None],
             jnp.zeros((D, 6), _F32)], axis=1)

    w1cat = jnp.stack([_wcat1(W1ub_s, a1ub_s, W1bu_d, a1bu_d),
                       _wcat1(W1bu_s, a1bu_s, W1ub_d, a1ub_d)])

    su, du, eu = _pad_edges(edge_index_u2b, edge_weight_u2b)
    sb, db, eb = _pad_edges(edge_index_b2u, edge_weight_b2u)
    ew_st = jnp.stack([eu, eb])

    bias1 = jnp.stack([b1ub, b1bu])
    bias2 = jnp.stack([jnp.tile(b2ub, 8), jnp.tile(b2bu, 8)])

    h1, s1, d1 = _tc1(x_user, x_badge, w1cat)
    agg1 = _make_sc(16)(h1, s1, d1, su, du, sb, db, ew_st, bias1)

    def _wcat2(ws, a_s, wd_other, a_d_other):
        return jnp.concatenate(
            [ws, (ws @ a_s)[:, None], (wd_other @ a_d_other)[:, None],
             jnp.zeros((16, 4), _F32)], axis=1)

    w2cat = jnp.stack([_wcat2(W2ub_s, a2ub_s, W2bu_d, a2bu_d),
                       _wcat2(W2bu_s, a2bu_s, W2ub_d, a2ub_d)])

    h2, s2, d2 = _tc2(agg1, w2cat)
    h2f = h2.reshape(2, NPAD * 2)
    agg2 = _make_sc(2)(h2f, s2, d2, su, du, sb, db, ew_st, bias2)

    badge2 = agg2[0].reshape(NPAD, 2)[:N]
    user2 = agg2[1].reshape(NPAD, 2)[:N]
    return (user2, badge2)


# speedup vs baseline: 98.4996x; 1.0116x over previous
import functools

import jax
import jax.numpy as jnp
from jax import lax
from jax.experimental import pallas as pl
from jax.experimental.pallas import tpu as pltpu
from jax.experimental.pallas import tpu_sc as plsc

N = 10000
D = 128
E = 320000
NPAD = 10240
RPT = NPAD // 16
NTILE = 16
CH = 157
EPT = CH * 128
EP = NTILE * EPT

_F32 = jnp.float32
_I32 = jnp.int32


def _tc1_body(xu_ref, xb_ref, w_ref, h_ref, s_ref, d_ref):
    zpad_h = jnp.zeros((NPAD - N, 16), _F32)
    zpad_1 = jnp.zeros((NPAD - N,), _F32)
    for t, x in ((0, xu_ref[...]), (1, xb_ref[...])):
        y = jnp.dot(x, w_ref[t], preferred_element_type=_F32)
        h_ref[t, :N] = y[:, :16]
        h_ref[t, N:] = zpad_h
        s_ref[t, 0, :N] = y[:, 16]
        s_ref[t, 0, N:] = zpad_1
        d_ref[t, 0, :N] = y[:, 17]
        d_ref[t, 0, N:] = zpad_1


_tc1 = pl.pallas_call(
    _tc1_body,
    out_shape=[
        jax.ShapeDtypeStruct((2, NPAD, 16), _F32),
        jax.ShapeDtypeStruct((2, 1, NPAD), _F32),
        jax.ShapeDtypeStruct((2, 1, NPAD), _F32),
    ],
)


def _tc2_body(agg_ref, w_ref, h_ref, s_ref, d_ref):
    x2 = jnp.maximum(agg_ref[0], 0.0)
    y = jnp.dot(x2, w_ref[0], preferred_element_type=_F32)
    h_ref[0] = y[:, :2]
    s_ref[0, 0] = y[:, 2]
    d_ref[0, 0] = y[:, 3]


_tc2 = pl.pallas_call(
    _tc2_body,
    grid=(2,),
    in_specs=[
        pl.BlockSpec((1, NPAD, 16), lambda t: (1 - t, 0, 0)),
        pl.BlockSpec((1, 16, 8), lambda t: (t, 0, 0)),
    ],
    out_specs=[
        pl.BlockSpec((1, NPAD, 2), lambda t: (t, 0, 0)),
        pl.BlockSpec((1, 1, NPAD), lambda t: (t, 0, 0)),
        pl.BlockSpec((1, 1, NPAD), lambda t: (t, 0, 0)),
    ],
    out_shape=[
        jax.ShapeDtypeStruct((2, NPAD, 2), _F32),
        jax.ShapeDtypeStruct((2, 1, NPAD), _F32),
        jax.ShapeDtypeStruct((2, 1, NPAD), _F32),
    ],
)


@functools.lru_cache(maxsize=None)
def _make_sc(h_dim):
    mesh = plsc.VectorSubcoreMesh(core_axis_name="c", subcore_axis_name="s",
                                  num_cores=2, num_subcores=NTILE)
    if h_dim == 16:
        out_type = jax.ShapeDtypeStruct((2, NPAD, 16), _F32)
        scratch = [
            pltpu.VMEM((CH, 128), _I32),
            pltpu.VMEM((CH, 128), _I32),
            pltpu.VMEM((4, 128), _F32),
            pltpu.VMEM((CH, 128), _F32),
            pltpu.VMEM((NPAD,), _F32),
            pltpu.VMEM((NPAD,), _F32),
            pltpu.VMEM((RPT, 16), _F32),
            pltpu.VMEM((4, 128, 16), _F32),
            pltpu.VMEM((16,), _F32),
            pltpu.VMEM((16, 16), _F32),
            pltpu.VMEM((16,), _F32),
            pltpu.VMEM((RPT,), _F32),
            pltpu.SemaphoreType.DMA,
            pltpu.SemaphoreType.DMA,
            pltpu.SemaphoreType.DMA((4,)),
            pltpu.SemaphoreType.DMA((4,)),
            pltpu.SemaphoreType.DMA((4,)),
            pltpu.MemorySpace.VMEM_SHARED((NPAD, 16), _F32),
            pltpu.MemorySpace.VMEM_SHARED((NPAD, 16), _F32),
            pltpu.MemorySpace.VMEM_SHARED((NPAD,), _F32),
            pltpu.MemorySpace.VMEM_SHARED((16, 16), _F32),
        ]
    else:
        out_type = jax.ShapeDtypeStruct((2, NPAD * 2), _F32)
        scratch = [
            pltpu.VMEM((CH, 128), _I32),
            pltpu.VMEM((CH, 128), _I32),
            pltpu.VMEM((2, 128), _F32),
            pltpu.VMEM((CH, 128), _F32),
            pltpu.VMEM((NPAD,), _F32),
            pltpu.VMEM((NPAD,), _F32),
            pltpu.VMEM((NPAD * 2,), _F32),
            pltpu.VMEM((2, 2, 128), _F32),
            pltpu.VMEM((2, 2, 128), _I32),
            pltpu.VMEM((2 * RPT,), _F32),
            pltpu.VMEM((16,), _F32),
            pltpu.VMEM((16, 16), _F32),
            pltpu.VMEM((16,), _F32),
            pltpu.VMEM((RPT,), _F32),
            pltpu.SemaphoreType.DMA,
            pltpu.SemaphoreType.DMA,
            pltpu.SemaphoreType.DMA((2,)),
            pltpu.SemaphoreType.DMA((2,)),
            pltpu.SemaphoreType.DMA((2,)),
            pltpu.MemorySpace.VMEM_SHARED((NPAD * 2,), _F32),
            pltpu.MemorySpace.VMEM_SHARED((NPAD,), _F32),
            pltpu.MemorySpace.VMEM_SHARED((16, 16), _F32),
        ]

    def body(h_hbm, s_hbm, d_hbm, su_hbm, du_hbm, sb_hbm, db_hbm, ew_hbm,
             b_hbm, out_hbm, *scr):
        if h_dim == 16:
            (srcv, dstv, ewrow, work, sbuf, dbuf, hstage, rows, maxv, maxall,
             bbuf, zv, psem, densem, gsem, ssem, esem, h_sp, out_sp, den_sp,
             max_sp) = scr
        else:
            (srcv, dstv, ewrow, work, sbuf, dbuf, h2d, vals, dstx2, ostage,
             maxv, maxall, bbuf, zv, psem, densem, gsem, ssem, esem, out_sp,
             den_sp, max_sp) = scr
        w = lax.axis_index("s")
        rel = lax.axis_index("c")
        r0 = w * RPT
        zero16 = jnp.zeros((16,), _F32)

        @pl.when(rel == 0)
        def _():
            pltpu.async_copy(su_hbm.at[w], srcv, psem)
            pltpu.async_copy(du_hbm.at[w], dstv, psem)

        @pl.when(rel == 1)
        def _():
            pltpu.async_copy(sb_hbm.at[w], srcv, psem)
            pltpu.async_copy(db_hbm.at[w], dstv, psem)
        pltpu.async_copy(s_hbm.at[rel, 0], sbuf, psem)
        pltpu.async_copy(d_hbm.at[1 - rel, 0], dbuf, psem)
        pltpu.async_copy(b_hbm.at[rel], bbuf, psem)

        if h_dim == 16:
            pltpu.async_copy(h_hbm.at[rel, pl.ds(r0, RPT)], hstage, psem)
        else:
            pltpu.async_copy(h_hbm.at[rel], h2d, psem)

        def _zv_row2(i, _):
            zv[pl.ds(i * 16, 16)] = zero16
            return 0
        lax.fori_loop(0, RPT // 16, _zv_row2, 0)
        pltpu.sync_copy(zv.at[pl.ds(0, RPT)], den_sp.at[pl.ds(r0, RPT)])

        @pl.when(rel == 0)
        def _():
            pltpu.make_async_copy(su_hbm.at[w], srcv, psem).wait()
            pltpu.make_async_copy(du_hbm.at[w], dstv, psem).wait()

        @pl.when(rel == 1)
        def _():
            pltpu.make_async_copy(sb_hbm.at[w], srcv, psem).wait()
            pltpu.make_async_copy(db_hbm.at[w], dstv, psem).wait()
        pltpu.make_async_copy(s_hbm.at[rel, 0], sbuf, psem).wait()
        pltpu.make_async_copy(d_hbm.at[1 - rel, 0], dbuf, psem).wait()
        pltpu.make_async_copy(b_hbm.at[rel], bbuf, psem).wait()

        neg = jnp.full((16,), -3.0e38, _F32)

        def _p1_row(cc, runmax):
            rm = runmax
            for k in range(8):
                si = srcv[cc, pl.ds(k * 16, 16)]
                di = dstv[cc, pl.ds(k * 16, 16)]
                sv = plsc.load_gather(sbuf, [si])
                dv = plsc.load_gather(dbuf, [di])
                logit = sv + dv
                logit = jnp.where(logit > 0.0, logit, 0.2 * logit)
                work[cc, pl.ds(k * 16, 16)] = logit
                rm = jnp.maximum(rm, logit)
            return rm

        runmax = lax.fori_loop(0, CH, _p1_row, neg)
        maxv[:] = jnp.broadcast_to(jnp.max(runmax), (16,))
        pltpu.sync_copy(maxv, max_sp.at[w])

        if h_dim == 16:
            pltpu.make_async_copy(h_hbm.at[rel, pl.ds(r0, RPT)], hstage,
                                  psem).wait()
            pltpu.sync_copy(hstage, h_sp.at[pl.ds(r0, RPT)])

            def _z_row(r, _):
                hstage[r, :] = zero16
                return 0
            lax.fori_loop(0, RPT, _z_row, 0)
            pltpu.sync_copy(hstage, out_sp.at[pl.ds(r0, RPT)])
        else:
            pltpu.make_async_copy(h_hbm.at[rel], h2d, psem).wait()

            def _z_row2(i, _):
                ostage[pl.ds(i * 16, 16)] = zero16
                return 0
            lax.fori_loop(0, (2 * RPT) // 16, _z_row2, 0)
            pltpu.sync_copy(ostage, out_sp.at[pl.ds(2 * r0, 2 * RPT)])
        plsc.subcore_barrier()

        pltpu.sync_copy(max_sp, maxall)

        def _mred(i, mm):
            return jnp.maximum(mm, maxall[i, :])
        m_glob = jnp.max(lax.fori_loop(0, 16, _mred, neg))

        def _p2_exp(cc):
            for k in range(8):
                logit = work[cc, pl.ds(k * 16, 16)]
                work[cc, pl.ds(k * 16, 16)] = jnp.exp(logit - m_glob)

        def _den_issue(cc):
            pltpu.async_copy(work.at[cc], den_sp.at[dstv.at[cc]], densem,
                             add=True)

        def _den_wait(cc):
            pltpu.make_async_copy(work.at[cc], den_sp.at[dstv.at[cc]],
                                  densem).wait()

        for cc in range(8):
            _p2_exp(cc)
            _den_issue(cc)

        def _p2_row(cc, _):
            _p2_exp(cc)
            _den_issue(cc)
            _den_wait(cc - 8)
            return 0
        lax.fori_loop(8, CH, _p2_row, 0)
        for cc in range(CH - 8, CH):
            _den_wait(cc)
        plsc.subcore_barrier()
        pltpu.sync_copy(den_sp, sbuf)

        iota = lax.iota(_I32, 16)
        zi = iota * 0

        def _coeff_row(cc, b):
            for k in range(8):
                ex = work[cc, pl.ds(k * 16, 16)]
                di = dstv[cc, pl.ds(k * 16, 16)]
                den = plsc.load_gather(sbuf, [di])
                alpha = ex / (den + 1e-16)
                work[cc, pl.ds(k * 16, 16)] = (
                    alpha * ewrow[b, pl.ds(k * 16, 16)])

        if h_dim == 16:
            def _g_issue(cc, b):
                pltpu.async_copy(h_sp.at[srcv.at[cc]], rows.at[b],
                                 gsem.at[b])
                pltpu.async_copy(ew_hbm.at[rel, w, cc], ewrow.at[b],
                                 esem.at[b])

            def _g_wait(cc, b):
                pltpu.make_async_copy(h_sp.at[srcv.at[cc]], rows.at[b],
                                      gsem.at[b]).wait()
                pltpu.make_async_copy(ew_hbm.at[rel, w, cc], ewrow.at[b],
                                      esem.at[b]).wait()

            def _s_issue(cc, b):
                pltpu.async_copy(rows.at[b], out_sp.at[dstv.at[cc]],
                                 ssem.at[b], add=True)

            def _s_wait(cc, b):
                pltpu.make_async_copy(rows.at[b], out_sp.at[dstv.at[cc]],
                                      ssem.at[b]).wait()

            def _step(cc, b, cc2, b2, do_swait):
                _g_wait(cc, b)
                _coeff_row(cc, b)

                def _scale_e(e, __):
                    ce = plsc.load_gather(work.at[cc], [zi + e])
                    rows[b, e, :] = rows[b, e, :] * ce
                    return 0
                lax.fori_loop(0, 128, _scale_e, 0, unroll=8)
                _s_issue(cc, b)
                if cc2 is not None:
                    if do_swait:
                        _s_wait(cc2 - 4, b2)
                    _g_issue(cc2, b2)

            _g_issue(0, 0)
            _g_issue(1, 1)
            _step(0, 0, 2, 2, False)
            _step(1, 1, 3, 3, False)

            def _p4_round(r, _):
                c0 = 4 * r + 2
                for j in range(4):
                    _step(c0 + j, (2 + j) % 4, c0 + j + 2, j, True)
                return 0
            lax.fori_loop(0, 38, _p4_round, 0)
            _step(154, 2, 156, 0, True)
            _step(155, 3, None, None, False)
            _step(156, 0, None, None, False)
            _s_wait(153, 1)
            _s_wait(154, 2)
            _s_wait(155, 3)
            _s_wait(156, 0)
        else:
            def _g_issue2(cc, b):
                pltpu.async_copy(ew_hbm.at[rel, w, cc], ewrow.at[b],
                                 gsem.at[b])

            def _g_wait2(cc, b):
                pltpu.make_async_copy(ew_hbm.at[rel, w, cc], ewrow.at[b],
                                      gsem.at[b]).wait()

            def _s_issue2(cc, b):
                pltpu.async_copy(vals.at[b, 0], out_sp.at[dstx2.at[b, 0]],
                                 ssem.at[b], add=True)
                pltpu.async_copy(vals.at[b, 1], out_sp.at[dstx2.at[b, 1]],
                                 ssem.at[b], add=True)

            def _s_wait2(cc, b):
                pltpu.make_async_copy(vals.at[b, 0],
                                      out_sp.at[dstx2.at[b, 0]],
                                      ssem.at[b]).wait()
                pltpu.make_async_copy(vals.at[b, 1],
                                      out_sp.at[dstx2.at[b, 1]],
                                      ssem.at[b]).wait()

            def _step2(cc, b, cc2, do_swait):
                _g_wait2(cc, b)
                _coeff_row(cc, b)
                for k in range(8):
                    si = srcv[cc, pl.ds(k * 16, 16)]
                    di = dstv[cc, pl.ds(k * 16, 16)]
                    co = work[cc, pl.ds(k * 16, 16)]
                    si2 = si * 2
                    vals[b, 0, pl.ds(k * 16, 16)] = \
                        plsc.load_gather(h2d, [si2]) * co
                    vals[b, 1, pl.ds(k * 16, 16)] = \
                        plsc.load_gather(h2d, [si2 + 1]) * co
                    dstx2[b, 0, pl.ds(k * 16, 16)] = di * 2
                    dstx2[b, 1, pl.ds(k * 16, 16)] = di * 2 + 1
                _s_issue2(cc, b)
                if cc2 is not None:
                    if do_swait:
                        _s_wait2(cc2 - 2, 1 - b)
                    _g_issue2(cc2, 1 - b)

            _g_issue2(0, 0)
            _step2(0, 0, 1, False)

            def _p4_round2(r, _):
                c0 = 2 * r + 1
                _step2(c0, 1, c0 + 1, True)
                _step2(c0 + 1, 0, c0 + 2, True)
                return 0
            lax.fori_loop(0, 77, _p4_round2, 0)
            _step2(155, 1, 156, True)
            _step2(156, 0, None, False)
            _s_wait2(155, 1)
            _s_wait2(156, 0)
        plsc.subcore_barrier()

        bv = bbuf[:]
        if h_dim == 16:
            pltpu.sync_copy(out_sp.at[pl.ds(r0, RPT)], hstage)

            def _b_row(r, _):
                hstage[r, :] = hstage[r, :] + bv
                return 0
            lax.fori_loop(0, RPT, _b_row, 0)
            pltpu.sync_copy(hstage, out_hbm.at[rel, pl.ds(r0, RPT)])
        else:
            o0 = 2 * r0
            pltpu.sync_copy(out_sp.at[pl.ds(o0, 2 * RPT)], ostage)

            def _b_row2(i, _):
                ostage[pl.ds(i * 16, 16)] = ostage[pl.ds(i * 16, 16)] + bv
                return 0
            lax.fori_loop(0, (2 * RPT) // 16, _b_row2, 0)
            pltpu.sync_copy(ostage, out_hbm.at[rel, pl.ds(o0, 2 * RPT)])

    return pl.kernel(
        body, out_type=out_type, mesh=mesh, scratch_types=scratch,
        compiler_params=pltpu.CompilerParams(needs_layout_passes=False,
                                             use_tc_tiling_on_sc=False))


def _pad_edges(ei, ew):
    src = jnp.concatenate([ei[0], jnp.full((EP - E,), N, _I32)])
    dst = jnp.concatenate([ei[1], jnp.full((EP - E,), N, _I32)])
    ewp = jnp.concatenate([ew, jnp.zeros((EP - E,), _F32)])
    return (src.reshape(NTILE, CH, 128), dst.reshape(NTILE, CH, 128),
            ewp.reshape(NTILE, CH, 128))


def kernel(x_user, x_badge, edge_index_u2b, edge_index_b2u,
           edge_weight_u2b, edge_weight_b2u,
           W1ub_s, W1ub_d, a1ub_s, a1ub_d, b1ub,
           W1bu_s, W1bu_d, a1bu_s, a1bu_d, b1bu,
           W2ub_s, W2ub_d, a2ub_s, a2ub_d, b2ub,
           W2bu_s, W2bu_d, a2bu_s, a2bu_d, b2bu):
    def _wcat1(ws, a_s, wd_other, a_d_other):
        return jnp.concatenate(
            [ws, (ws @ a_s)[:, None], (wd_other @ a_d_other)[:, None],
             jnp.zeros((D, 6), _F32)], axis=1)

    w1cat = jnp.stack([_wcat1(W1ub_s, a1ub_s, W1bu_d, a1bu_d),
                       _wcat1(W1bu_s, a1bu_s, W1ub_d, a1ub_d)])

    su, du, eu = _pad_edges(edge_index_u2b, edge_weight_u2b)
    sb, db, eb = _pad_edges(edge_index_b2u, edge_weight_b2u)
    ew_st = jnp.stack([eu, eb])

    bias1 = jnp.stack([b1ub, b1bu])
    bias2 = jnp.stack([jnp.tile(b2ub, 8), jnp.tile(b2bu, 8)])

    h1, s1, d1 = _tc1(x_user, x_badge, w1cat)
    agg1 = _make_sc(16)(h1, s1, d1, su, du, sb, db, ew_st, bias1)

    def _wcat2(ws, a_s, wd_other, a_d_other):
        return jnp.concatenate(
            [ws, (ws @ a_s)[:, None], (wd_other @ a_d_other)[:, None],
             jnp.zeros((16, 4), _F32)], axis=1)

    w2cat = jnp.stack([_wcat2(W2ub_s, a2ub_s, W2bu_d, a2bu_d),
                       _wcat2(W2bu_s, a2bu_s, W2ub_d, a2ub_d)])

    h2, s2, d2 = _tc2(agg1, w2cat)
    h2f = h2.reshape(2, NPAD * 2)
    agg2 = _make_sc(2)(h2f, s2, d2, su, du, sb, db, ew_st, bias2)

    badge2 = agg2[0].reshape(NPAD, 2)[:N]
    user2 = agg2[1].reshape(NPAD, 2)[:N]
    return (user2, badge2)
